# baseline (speedup 1.0000x reference)
import functools

import jax
import jax.numpy as jnp
from jax import lax
from jax.experimental import pallas as pl
from jax.experimental.pallas import tpu as pltpu
from jax.experimental.pallas import tpu_sc as plsc

B, N, K = 8, 1024, 8
D_PT, D_PH, D_OBJ = 256, 128, 256
D_LIN, D_LH, D_LANG = 768, 256, 256
D_RH = 128
D_FUS = 512
DIAM = 5.0

TA = 256
TB = 256
TN = 128

SC_NC, SC_NS = 2, 16
SC_NW = SC_NC * SC_NS
BH = B // 2
SC_ROWS = BH * N * K
SC_RPW = SC_ROWS // SC_NW
SC_CH = 128
SC_NCH = SC_RPW // SC_CH


def _enc_body(pts_ref, cen_ref, siz_ref, mf_ref, cent_ref, mrow_ref,
              pe_w1_ref, pe_b1_ref, pe_w2_ref, pe_b2_ref,
              pr_w1_ref, pr_w2_ref, sp_w_ref,
              obj_ref, g1_ref, tab_ref, ub_ref,
              nbr_ref, nbrg_ref, dval_ref):
    b = pl.program_id(0)
    i = pl.program_id(1)
    cb = cen_ref[0]
    ct = cent_ref[0]
    dx = cb[:, 0:1] - ct[0:1, :]
    dy = cb[:, 1:2] - ct[1:2, :]
    dz = cb[:, 2:3] - ct[2:3, :]
    dist = jnp.sqrt(dx * dx + dy * dy + dz * dz + 1e-12) / DIAM
    col = lax.broadcasted_iota(jnp.int32, (TA, N), 1)
    row = i * TA + lax.broadcasted_iota(jnp.int32, (TA, N), 0)
    valid = (mrow_ref[0] > 0.0) & (col != row)
    dm = jnp.where(valid, dist, jnp.inf)
    for k in range(K):
        minv = jnp.min(dm, axis=1, keepdims=True)
        idx = jnp.min(jnp.where(dm == minv, col, N), axis=1, keepdims=True)
        nbr_ref[0, :, k:k + 1] = idx
        nbrg_ref[0, :, k:k + 1] = idx + b * N
        dval_ref[0, :, k:k + 1] = minv
        dm = jnp.where(col == idx, jnp.inf, dm)

    pts = pts_ref[0]
    h = jnp.maximum(jnp.dot(pts, pe_w1_ref[...],
                            preferred_element_type=jnp.float32)
                    + pe_b1_ref[...], 0.0)
    obj = (jnp.dot(h, pe_w2_ref[...], preferred_element_type=jnp.float32)
           + pe_b2_ref[...]) * mf_ref[0]
    obj_ref[0] = obj
    g1_ref[0] = jnp.dot(obj, pr_w1_ref[...], preferred_element_type=jnp.float32)
    cw = jnp.dot(cen_ref[0], sp_w_ref[0:3, :],
                 preferred_element_type=jnp.float32) * (1.0 / DIAM)
    g2 = jnp.dot(obj, pr_w2_ref[...], preferred_element_type=jnp.float32)
    ua = cw + jnp.dot(siz_ref[0], sp_w_ref[7:10, :],
                      preferred_element_type=jnp.float32)
    g2i = lax.bitcast_convert_type(g2, jnp.int32)
    uai = lax.bitcast_convert_type(ua, jnp.int32)
    hi = (g2i + 0x8000) & jnp.int32(-65536)
    lo = jnp.right_shift(uai + 0x8000, 16) & 0xFFFF
    tab_ref[0] = hi | lo
    ub_ref[0] = jnp.dot(siz_ref[0], sp_w_ref[4:7, :],
                        preferred_element_type=jnp.float32) - cw


def _encode(points, centers, sizes, maskf, centers_t, maskf_row,
            pe_w1, pe_b1, pe_w2, pe_b2, pr_w1, pr_w2, sp_w):
    grid = (B, N // TA)
    return pl.pallas_call(
        _enc_body,
        grid=grid,
        in_specs=[
            pl.BlockSpec((1, TA, D_PT), lambda b, i: (b, i, 0)),
            pl.BlockSpec((1, TA, 3), lambda b, i: (b, i, 0)),
            pl.BlockSpec((1, TA, 3), lambda b, i: (b, i, 0)),
            pl.BlockSpec((1, TA, 1), lambda b, i: (b, i, 0)),
            pl.BlockSpec((1, 3, N), lambda b, i: (b, 0, 0)),
            pl.BlockSpec((1, 1, N), lambda b, i: (b, 0, 0)),
            pl.BlockSpec((D_PT, D_PH), lambda b, i: (0, 0)),
            pl.BlockSpec((1, D_PH), lambda b, i: (0, 0)),
            pl.BlockSpec((D_PH, D_OBJ), lambda b, i: (0, 0)),
            pl.BlockSpec((1, D_OBJ), lambda b, i: (0, 0)),
            pl.BlockSpec((D_OBJ, D_RH), lambda b, i: (0, 0)),
            pl.BlockSpec((D_OBJ, D_RH), lambda b, i: (0, 0)),
            pl.BlockSpec((10, D_RH), lambda b, i: (0, 0)),
        ],
        out_specs=[
            pl.BlockSpec((1, TA, D_OBJ), lambda b, i: (b, i, 0)),
            pl.BlockSpec((1, TA, D_RH), lambda b, i: (b, i, 0)),
            pl.BlockSpec((1, TA, D_RH), lambda b, i: (b, i, 0)),
            pl.BlockSpec((1, TA, D_RH), lambda b, i: (b, i, 0)),
            pl.BlockSpec((1, TA, K), lambda b, i: (b, i, 0)),
            pl.BlockSpec((1, TA, K), lambda b, i: (b, i, 0)),
            pl.BlockSpec((1, TA, K), lambda b, i: (b, i, 0)),
        ],
        out_shape=[
            jax.ShapeDtypeStruct((B, N, D_OBJ), jnp.float32),
            jax.ShapeDtypeStruct((B, N, D_RH), jnp.float32),
            jax.ShapeDtypeStruct((B, N, D_RH), jnp.int32),
            jax.ShapeDtypeStruct((B, N, D_RH), jnp.float32),
            jax.ShapeDtypeStruct((B, N, K), jnp.int32),
            jax.ShapeDtypeStruct((B, N, K), jnp.int32),
            jax.ShapeDtypeStruct((B, N, K), jnp.float32),
        ],
    )(points, centers, sizes, maskf, centers_t, maskf_row,
      pe_w1, pe_b1, pe_w2, pe_b2, pr_w1, pr_w2, sp_w)


def _sc_gather_body(idx_hbm, tab_hbm, out_hbm, idx_v,
                    buf0, buf1, gs0, gs1, ws0, ws1):
    wid = lax.axis_index("s") * SC_NC + lax.axis_index("c")
    base = wid * SC_RPW
    pltpu.sync_copy(idx_hbm.at[pl.ds(base, SC_RPW)], idx_v)
    bufs = (buf0, buf1)
    gsem = (gs0, gs1)
    wsem = (ws0, ws1)
    gops = [None, None]
    wops = [None, None]
    for ci in range(SC_NCH):
        s = ci % 2
        if wops[s] is not None:
            wops[s].wait()
        gops[s] = pltpu.async_copy(
            tab_hbm.at[idx_v.at[pl.ds(ci * SC_CH, SC_CH)]], bufs[s], gsem[s])
        p = 1 - s
        if gops[p] is not None:
            gops[p].wait()
            wops[p] = pltpu.async_copy(
                bufs[p], out_hbm.at[pl.ds(base + (ci - 1) * SC_CH, SC_CH)],
                wsem[p])
    last = (SC_NCH - 1) % 2
    gops[last].wait()
    wops[last] = pltpu.async_copy(
        bufs[last], out_hbm.at[pl.ds(base + (SC_NCH - 1) * SC_CH, SC_CH)],
        wsem[last])
    wops[1 - last].wait()
    wops[last].wait()


@functools.cache
def _sc_gather_kernel():
    return pl.kernel(
        _sc_gather_body,
        out_type=jax.ShapeDtypeStruct((SC_ROWS, D_RH), jnp.int32),
        mesh=plsc.VectorSubcoreMesh(core_axis_name="c", subcore_axis_name="s",
                                    num_cores=SC_NC, num_subcores=SC_NS),
        scratch_types=[
            pltpu.VMEM((SC_RPW,), jnp.int32),
            pltpu.VMEM((SC_CH, D_RH), jnp.int32),
            pltpu.VMEM((SC_CH, D_RH), jnp.int32),
            pltpu.SemaphoreType.DMA,
            pltpu.SemaphoreType.DMA,
            pltpu.SemaphoreType.DMA,
            pltpu.SemaphoreType.DMA,
        ],
    )


def _sc_gather(nbrg_flat, tab):
    return _sc_gather_kernel()(nbrg_flat, tab)


def _rel_body(obj_ref, g1_ref, ub_ref, mf_ref, tabg_ref, dk_ref,
              sp_w3_ref, sp_b_ref, pr_w3_ref, pr_b_ref, sc_w_ref,
              vl_w_ref, vl_b_ref, fu_w1a_ref, langc_ref, fu_b1_ref,
              fu_w2_ref, fu_b2_ref, cl_w_ref, cl_b_ref,
              wgt_ref, enh_ref, log_ref):
    tabg3 = tabg_ref[0].reshape(TN, K, D_RH)
    g2j3 = lax.bitcast_convert_type(tabg3 & jnp.int32(-65536), jnp.float32)
    uaj3 = lax.bitcast_convert_type(jnp.left_shift(tabg3, 16), jnp.float32)
    hdk = jnp.dot(dk_ref[0], sp_w3_ref[...],
                  preferred_element_type=jnp.float32)
    hsp3 = jnp.maximum(
        uaj3
        + ub_ref[0].reshape(TN, 1, D_RH)
        + hdk.reshape(TN, K, D_RH)
        + sp_b_ref[...].reshape(1, 1, D_RH), 0.0)

    hp = jnp.dot(hsp3.reshape(TN * K, D_RH), pr_w3_ref[...],
                 preferred_element_type=jnp.float32)
    pair3 = jnp.maximum(
        g1_ref[0].reshape(TN, 1, D_RH)
        + g2j3
        + hp.reshape(TN, K, D_RH)
        + pr_b_ref[...].reshape(1, 1, D_RH), 0.0)
    pair2 = pair3.reshape(TN * K, D_RH)

    w_rep = jnp.broadcast_to(sc_w_ref[0:D_RH, :], (D_RH, D_RH))
    score = jnp.dot(pair2, w_rep,
                    preferred_element_type=jnp.float32).reshape(TN, K, D_RH)
    m = jnp.max(score, axis=1, keepdims=True)
    e = jnp.exp(score - m)
    s = jnp.sum(e, axis=1, keepdims=True)

    score_c = jnp.dot(pair2, sc_w_ref[0:D_RH, :],
                      preferred_element_type=jnp.float32).reshape(TN, K, 1)
    mc = jnp.max(score_c, axis=1, keepdims=True)
    ec = jnp.exp(score_c - mc)
    sc = jnp.sum(ec, axis=1, keepdims=True)
    wgt_ref[0] = (ec / sc).reshape(TN * K, 1)

    pbar = jnp.sum(e * pair3, axis=1) / s[:, 0, :]
    ctx = jnp.dot(pbar, vl_w_ref[...],
                  preferred_element_type=jnp.float32) + vl_b_ref[...]
    mf = mf_ref[0]
    enh = (obj_ref[0] + ctx) * mf
    enh_ref[0] = enh

    f1 = jnp.maximum(jnp.dot(enh, fu_w1a_ref[...],
                             preferred_element_type=jnp.float32)
                     + langc_ref[0] + fu_b1_ref[...], 0.0)
    f2 = jnp.maximum(jnp.dot(f1, fu_w2_ref[...],
                             preferred_element_type=jnp.float32)
                     + fu_b2_ref[...], 0.0)
    logit = jnp.dot(f2, cl_w_ref[...],
                    preferred_element_type=jnp.float32) + cl_b_ref[...]
    log_ref[0] = jnp.where(mf > 0.0, logit, -jnp.inf)


def _relation(obj, g1, ub, maskf, tabg, dk, sp_w3, sp_b2, pr_w3, pr_b2,
              sc_w, vl_w, vl_b2, fu_w1a, langc, fu_b12, fu_w2, fu_b22,
              cl_w, cl_b2):
    grid = (BH, N // TN)
    return pl.pallas_call(
        _rel_body,
        grid=grid,
        in_specs=[
            pl.BlockSpec((1, TN, D_OBJ), lambda b, i: (b, i, 0)),
            pl.BlockSpec((1, TN, D_RH), lambda b, i: (b, i, 0)),
            pl.BlockSpec((1, TN, D_RH), lambda b, i: (b, i, 0)),
            pl.BlockSpec((1, TN, 1), lambda b, i: (b, i, 0)),
            pl.BlockSpec((1, TN * K, D_RH), lambda b, i: (b, i, 0)),
            pl.BlockSpec((1, TN * K, 1), lambda b, i: (b, i, 0)),
            pl.BlockSpec((1, D_RH), lambda b, i: (0, 0)),
            pl.BlockSpec((1, D_RH), lambda b, i: (0, 0)),
            pl.BlockSpec((D_RH, D_RH), lambda b, i: (0, 0)),
            pl.BlockSpec((1, D_RH), lambda b, i: (0, 0)),
            pl.BlockSpec((D_RH + D_LANG, 1), lambda b, i: (0, 0)),
            pl.BlockSpec((D_RH, D_OBJ), lambda b, i: (0, 0)),
            pl.BlockSpec((1, D_OBJ), lambda b, i: (0, 0)),
            pl.BlockSpec((D_OBJ, D_FUS), lambda b, i: (0, 0)),
            pl.BlockSpec((1, 1, D_FUS), lambda b, i: (b, 0, 0)),
            pl.BlockSpec((1, D_FUS), lambda b, i: (0, 0)),
            pl.BlockSpec((D_FUS, D_FUS), lambda b, i: (0, 0)),
            pl.BlockSpec((1, D_FUS), lambda b, i: (0, 0)),
            pl.BlockSpec((D_FUS, 1), lambda b, i: (0, 0)),
            pl.BlockSpec((1, 1), lambda b, i: (0, 0)),
        ],
        out_specs=[
            pl.BlockSpec((1, TN * K, 1), lambda b, i: (b, i, 0)),
            pl.BlockSpec((1, TN, D_OBJ), lambda b, i: (b, i, 0)),
            pl.BlockSpec((1, TN, 1), lambda b, i: (b, i, 0)),
        ],
        out_shape=[
            jax.ShapeDtypeStruct((BH, N * K, 1), jnp.float32),
            jax.ShapeDtypeStruct((BH, N, D_OBJ), jnp.float32),
            jax.ShapeDtypeStruct((BH, N, 1), jnp.float32),
        ],
    )(obj, g1, ub, maskf, tabg, dk, sp_w3, sp_b2, pr_w3, pr_b2, sc_w,
      vl_w, vl_b2, fu_w1a, langc, fu_b12, fu_w2, fu_b22, cl_w, cl_b2)


def _lang_body(txt_ref, le_w1_ref, le_b1_ref, le_w2_ref, le_b2_ref,
               fu_w1b_ref, lang_ref, langc_ref):
    h = jnp.maximum(jnp.dot(txt_ref[...], le_w1_ref[...],
                            preferred_element_type=jnp.float32)
                    + le_b1_ref[...], 0.0)
    lang = jnp.dot(h, le_w2_ref[...],
                   preferred_element_type=jnp.float32) + le_b2_ref[...]
    lang_ref[...] = lang
    langc_ref[...] = jnp.dot(lang, fu_w1b_ref[...],
                             preferred_element_type=jnp.float32)


def _lang(text_features, le_w1, le_b12, le_w2, le_b22, fu_w1b):
    return pl.pallas_call(
        _lang_body,
        out_shape=[
            jax.ShapeDtypeStruct((B, D_LANG), jnp.float32),
            jax.ShapeDtypeStruct((B, D_FUS), jnp.float32),
        ],
    )(text_features, le_w1, le_b12, le_w2, le_b22, fu_w1b)


def kernel(points, object_mask, text_features, centers, sizes, pe_w1, pe_b1,
           pe_w2, pe_b2, le_w1, le_b1, le_w2, le_b2, sp_w, sp_b, pr_w, pr_b,
           sc_w, sc_b, vl_w, vl_b, fu_w1, fu_b1, fu_w2, fu_b2, cl_w, cl_b):
    maskf = object_mask.astype(jnp.float32)[:, :, None]
    maskf_row = object_mask.astype(jnp.float32)[:, None, :]
    centers_t = jnp.transpose(centers, (0, 2, 1))

    obj, g1, tab, ub, nbr, nbrg, dval = _encode(
        points, centers, sizes, maskf, centers_t, maskf_row,
        pe_w1, pe_b1[None, :], pe_w2, pe_b2[None, :],
        pr_w[0:D_OBJ], pr_w[D_OBJ:2 * D_OBJ], sp_w)

    tab2 = tab.reshape(B * N, D_RH)
    tabg0 = _sc_gather(nbrg[0:BH].reshape(-1), tab2)
    tabg1 = _sc_gather(nbrg[BH:].reshape(-1), tab2)

    lang, langc = _lang(text_features, le_w1, le_b1[None, :], le_w2,
                        le_b2[None, :], fu_w1[D_OBJ:])

    halves = []
    for h, tabg in ((0, tabg0), (1, tabg1)):
        s = slice(h * BH, (h + 1) * BH)
        halves.append(_relation(
            obj[s], g1[s], ub[s], maskf[s], tabg.reshape(BH, N * K, D_RH),
            dval[s].reshape(BH, N * K, 1), sp_w[3:4], sp_b[None, :],
            pr_w[2 * D_OBJ:], pr_b[None, :], sc_w, vl_w, vl_b[None, :],
            fu_w1[0:D_OBJ], langc[s, None, :], fu_b1[None, :], fu_w2,
            fu_b2[None, :],
            cl_w, cl_b[None, :]))

    wgt = jnp.concatenate([halves[0][0], halves[1][0]], 0).reshape(B, N, K)
    enhanced = jnp.concatenate([halves[0][1], halves[1][1]], 0)
    logits = jnp.concatenate([halves[0][2], halves[1][2]], 0)[:, :, 0]
    return (logits, enhanced, obj, lang, wgt, nbr)


# speedup vs baseline: 1.0994x; 1.0994x over previous
import functools

import jax
import jax.numpy as jnp
from jax import lax
from jax.experimental import pallas as pl
from jax.experimental.pallas import tpu as pltpu
from jax.experimental.pallas import tpu_sc as plsc

B, N, K = 8, 1024, 8
D_PT, D_PH, D_OBJ = 256, 128, 256
D_LIN, D_LH, D_LANG = 768, 256, 256
D_RH = 128
D_FUS = 512
DIAM = 5.0

TA = 256
TB = 256
TN = 128

SC_NC, SC_NS = 2, 16
SC_NW = SC_NC * SC_NS
SC_ROWS = B * N * K
SC_RPW = SC_ROWS // SC_NW
SC_CH = 128
SC_NCH = SC_RPW // SC_CH


def _enc_body(pts_ref, cen_ref, siz_ref, mf_ref, cent_ref, mrow_ref,
              pe_w1_ref, pe_b1_ref, pe_w2_ref, pe_b2_ref,
              pr_w1_ref, pr_w2_ref, sp_w_ref,
              obj_ref, g1_ref, tab_ref, ub_ref,
              nbr_ref, nbrg_ref, dval_ref):
    b = pl.program_id(0)
    i = pl.program_id(1)
    cb = cen_ref[0]
    ct = cent_ref[0]
    dx = cb[:, 0:1] - ct[0:1, :]
    dy = cb[:, 1:2] - ct[1:2, :]
    dz = cb[:, 2:3] - ct[2:3, :]
    dist = jnp.sqrt(dx * dx + dy * dy + dz * dz + 1e-12) / DIAM
    col = lax.broadcasted_iota(jnp.int32, (TA, N), 1)
    row = i * TA + lax.broadcasted_iota(jnp.int32, (TA, N), 0)
    valid = (mrow_ref[0] > 0.0) & (col != row)
    dm = jnp.where(valid, dist, jnp.inf)
    for k in range(K):
        minv = jnp.min(dm, axis=1, keepdims=True)
        idx = jnp.min(jnp.where(dm == minv, col, N), axis=1, keepdims=True)
        nbr_ref[0, :, k:k + 1] = idx
        nbrg_ref[0, :, k:k + 1] = idx + b * N
        dval_ref[0, :, k:k + 1] = minv
        dm = jnp.where(col == idx, jnp.inf, dm)

    pts = pts_ref[0]
    h = jnp.maximum(jnp.dot(pts, pe_w1_ref[...],
                            preferred_element_type=jnp.float32)
                    + pe_b1_ref[...], 0.0)
    obj = (jnp.dot(h, pe_w2_ref[...], preferred_element_type=jnp.float32)
           + pe_b2_ref[...]) * mf_ref[0]
    obj_ref[0] = obj
    g1_ref[0] = jnp.dot(obj, pr_w1_ref[...], preferred_element_type=jnp.float32)
    cw = jnp.dot(cen_ref[0], sp_w_ref[0:3, :],
                 preferred_element_type=jnp.float32) * (1.0 / DIAM)
    g2 = jnp.dot(obj, pr_w2_ref[...], preferred_element_type=jnp.float32)
    ua = cw + jnp.dot(siz_ref[0], sp_w_ref[7:10, :],
                      preferred_element_type=jnp.float32)
    g2i = lax.bitcast_convert_type(g2, jnp.int32)
    uai = lax.bitcast_convert_type(ua, jnp.int32)
    hi = (g2i + 0x8000) & jnp.int32(-65536)
    lo = jnp.right_shift(uai + 0x8000, 16) & 0xFFFF
    tab_ref[0] = hi | lo
    ub_ref[0] = jnp.dot(siz_ref[0], sp_w_ref[4:7, :],
                        preferred_element_type=jnp.float32) - cw


def _encode(points, centers, sizes, maskf, centers_t, maskf_row,
            pe_w1, pe_b1, pe_w2, pe_b2, pr_w1, pr_w2, sp_w):
    grid = (B, N // TA)
    return pl.pallas_call(
        _enc_body,
        grid=grid,
        in_specs=[
            pl.BlockSpec((1, TA, D_PT), lambda b, i: (b, i, 0)),
            pl.BlockSpec((1, TA, 3), lambda b, i: (b, i, 0)),
            pl.BlockSpec((1, TA, 3), lambda b, i: (b, i, 0)),
            pl.BlockSpec((1, TA, 1), lambda b, i: (b, i, 0)),
            pl.BlockSpec((1, 3, N), lambda b, i: (b, 0, 0)),
            pl.BlockSpec((1, 1, N), lambda b, i: (b, 0, 0)),
            pl.BlockSpec((D_PT, D_PH), lambda b, i: (0, 0)),
            pl.BlockSpec((1, D_PH), lambda b, i: (0, 0)),
            pl.BlockSpec((D_PH, D_OBJ), lambda b, i: (0, 0)),
            pl.BlockSpec((1, D_OBJ), lambda b, i: (0, 0)),
            pl.BlockSpec((D_OBJ, D_RH), lambda b, i: (0, 0)),
            pl.BlockSpec((D_OBJ, D_RH), lambda b, i: (0, 0)),
            pl.BlockSpec((10, D_RH), lambda b, i: (0, 0)),
        ],
        out_specs=[
            pl.BlockSpec((1, TA, D_OBJ), lambda b, i: (b, i, 0)),
            pl.BlockSpec((1, TA, D_RH), lambda b, i: (b, i, 0)),
            pl.BlockSpec((1, TA, D_RH), lambda b, i: (b, i, 0)),
            pl.BlockSpec((1, TA, D_RH), lambda b, i: (b, i, 0)),
            pl.BlockSpec((1, TA, K), lambda b, i: (b, i, 0)),
            pl.BlockSpec((1, TA, K), lambda b, i: (b, i, 0)),
            pl.BlockSpec((1, TA, K), lambda b, i: (b, i, 0)),
        ],
        out_shape=[
            jax.ShapeDtypeStruct((B, N, D_OBJ), jnp.float32),
            jax.ShapeDtypeStruct((B, N, D_RH), jnp.float32),
            jax.ShapeDtypeStruct((B, N, D_RH), jnp.int32),
            jax.ShapeDtypeStruct((B, N, D_RH), jnp.float32),
            jax.ShapeDtypeStruct((B, N, K), jnp.int32),
            jax.ShapeDtypeStruct((B, N, K), jnp.int32),
            jax.ShapeDtypeStruct((B, N, K), jnp.float32),
        ],
    )(points, centers, sizes, maskf, centers_t, maskf_row,
      pe_w1, pe_b1, pe_w2, pe_b2, pr_w1, pr_w2, sp_w)


def _sc_gather_body(idx_hbm, tab_hbm, out_hbm, idx_v,
                    buf0, buf1, gs0, gs1, ws0, ws1):
    wid = lax.axis_index("s") * SC_NC + lax.axis_index("c")
    base = wid * SC_RPW
    pltpu.sync_copy(idx_hbm.at[pl.ds(base, SC_RPW)], idx_v)
    bufs = (buf0, buf1)
    gsem = (gs0, gs1)
    wsem = (ws0, ws1)
    gops = [None, None]
    wops = [None, None]
    for ci in range(SC_NCH):
        s = ci % 2
        if wops[s] is not None:
            wops[s].wait()
        gops[s] = pltpu.async_copy(
            tab_hbm.at[idx_v.at[pl.ds(ci * SC_CH, SC_CH)]], bufs[s], gsem[s])
        p = 1 - s
        if gops[p] is not None:
            gops[p].wait()
            wops[p] = pltpu.async_copy(
                bufs[p], out_hbm.at[pl.ds(base + (ci - 1) * SC_CH, SC_CH)],
                wsem[p])
    last = (SC_NCH - 1) % 2
    gops[last].wait()
    wops[last] = pltpu.async_copy(
        bufs[last], out_hbm.at[pl.ds(base + (SC_NCH - 1) * SC_CH, SC_CH)],
        wsem[last])
    wops[1 - last].wait()
    wops[last].wait()


@functools.cache
def _sc_gather_kernel():
    return pl.kernel(
        _sc_gather_body,
        out_type=jax.ShapeDtypeStruct((SC_ROWS, D_RH), jnp.int32),
        mesh=plsc.VectorSubcoreMesh(core_axis_name="c", subcore_axis_name="s",
                                    num_cores=SC_NC, num_subcores=SC_NS),
        scratch_types=[
            pltpu.VMEM((SC_RPW,), jnp.int32),
            pltpu.VMEM((SC_CH, D_RH), jnp.int32),
            pltpu.VMEM((SC_CH, D_RH), jnp.int32),
            pltpu.SemaphoreType.DMA,
            pltpu.SemaphoreType.DMA,
            pltpu.SemaphoreType.DMA,
            pltpu.SemaphoreType.DMA,
        ],
    )


def _sc_gather(nbrg_flat, tab):
    return _sc_gather_kernel()(nbrg_flat, tab)


def _rel_body(obj_ref, g1_ref, ub_ref, mf_ref, tabg_ref, dk_ref,
              sp_w3_ref, sp_b_ref, pr_w3_ref, pr_b_ref, sc_w_ref,
              vl_w_ref, vl_b_ref, fu_w1a_ref, langc_ref, fu_b1_ref,
              fu_w2_ref, fu_b2_ref, cl_w_ref, cl_b_ref,
              wgt_ref, enh_ref, log_ref):
    tabg3 = tabg_ref[0].reshape(TN, K, D_RH)
    g2j3 = lax.bitcast_convert_type(tabg3 & jnp.int32(-65536), jnp.float32)
    uaj3 = lax.bitcast_convert_type(jnp.left_shift(tabg3, 16), jnp.float32)
    hdk = jnp.dot(dk_ref[0], sp_w3_ref[...],
                  preferred_element_type=jnp.float32)
    hsp3 = jnp.maximum(
        uaj3
        + ub_ref[0].reshape(TN, 1, D_RH)
        + hdk.reshape(TN, K, D_RH)
        + sp_b_ref[...].reshape(1, 1, D_RH), 0.0)

    hp = jnp.dot(hsp3.reshape(TN * K, D_RH), pr_w3_ref[...],
                 preferred_element_type=jnp.float32)
    pair3 = jnp.maximum(
        g1_ref[0].reshape(TN, 1, D_RH)
        + g2j3
        + hp.reshape(TN, K, D_RH)
        + pr_b_ref[...].reshape(1, 1, D_RH), 0.0)
    pair2 = pair3.reshape(TN * K, D_RH)

    w_rep = jnp.broadcast_to(sc_w_ref[0:D_RH, :], (D_RH, D_RH))
    score = jnp.dot(pair2, w_rep,
                    preferred_element_type=jnp.float32).reshape(TN, K, D_RH)
    m = jnp.max(score, axis=1, keepdims=True)
    e = jnp.exp(score - m)
    s = jnp.sum(e, axis=1, keepdims=True)

    score_c = jnp.dot(pair2, sc_w_ref[0:D_RH, :],
                      preferred_element_type=jnp.float32).reshape(TN, K, 1)
    mc = jnp.max(score_c, axis=1, keepdims=True)
    ec = jnp.exp(score_c - mc)
    sc = jnp.sum(ec, axis=1, keepdims=True)
    wgt_ref[0] = (ec / sc).reshape(TN * K, 1)

    pbar = jnp.sum(e * pair3, axis=1) / s[:, 0, :]
    ctx = jnp.dot(pbar, vl_w_ref[...],
                  preferred_element_type=jnp.float32) + vl_b_ref[...]
    mf = mf_ref[0]
    enh = (obj_ref[0] + ctx) * mf
    enh_ref[0] = enh

    f1 = jnp.maximum(jnp.dot(enh, fu_w1a_ref[...],
                             preferred_element_type=jnp.float32)
                     + langc_ref[0] + fu_b1_ref[...], 0.0)
    f2 = jnp.maximum(jnp.dot(f1, fu_w2_ref[...],
                             preferred_element_type=jnp.float32)
                     + fu_b2_ref[...], 0.0)
    logit = jnp.dot(f2, cl_w_ref[...],
                    preferred_element_type=jnp.float32) + cl_b_ref[...]
    log_ref[0] = jnp.where(mf > 0.0, logit, -jnp.inf)


def _relation(obj, g1, ub, maskf, tabg, dk, sp_w3, sp_b2, pr_w3, pr_b2,
              sc_w, vl_w, vl_b2, fu_w1a, langc, fu_b12, fu_w2, fu_b22,
              cl_w, cl_b2):
    grid = (B, N // TN)
    return pl.pallas_call(
        _rel_body,
        grid=grid,
        in_specs=[
            pl.BlockSpec((1, TN, D_OBJ), lambda b, i: (b, i, 0)),
            pl.BlockSpec((1, TN, D_RH), lambda b, i: (b, i, 0)),
            pl.BlockSpec((1, TN, D_RH), lambda b, i: (b, i, 0)),
            pl.BlockSpec((1, TN, 1), lambda b, i: (b, i, 0)),
            pl.BlockSpec((1, TN * K, D_RH), lambda b, i: (b, i, 0)),
            pl.BlockSpec((1, TN * K, 1), lambda b, i: (b, i, 0)),
            pl.BlockSpec((1, D_RH), lambda b, i: (0, 0)),
            pl.BlockSpec((1, D_RH), lambda b, i: (0, 0)),
            pl.BlockSpec((D_RH, D_RH), lambda b, i: (0, 0)),
            pl.BlockSpec((1, D_RH), lambda b, i: (0, 0)),
            pl.BlockSpec((D_RH + D_LANG, 1), lambda b, i: (0, 0)),
            pl.BlockSpec((D_RH, D_OBJ), lambda b, i: (0, 0)),
            pl.BlockSpec((1, D_OBJ), lambda b, i: (0, 0)),
            pl.BlockSpec((D_OBJ, D_FUS), lambda b, i: (0, 0)),
            pl.BlockSpec((1, 1, D_FUS), lambda b, i: (b, 0, 0)),
            pl.BlockSpec((1, D_FUS), lambda b, i: (0, 0)),
            pl.BlockSpec((D_FUS, D_FUS), lambda b, i: (0, 0)),
            pl.BlockSpec((1, D_FUS), lambda b, i: (0, 0)),
            pl.BlockSpec((D_FUS, 1), lambda b, i: (0, 0)),
            pl.BlockSpec((1, 1), lambda b, i: (0, 0)),
        ],
        out_specs=[
            pl.BlockSpec((1, TN * K, 1), lambda b, i: (b, i, 0)),
            pl.BlockSpec((1, TN, D_OBJ), lambda b, i: (b, i, 0)),
            pl.BlockSpec((1, TN, 1), lambda b, i: (b, i, 0)),
        ],
        out_shape=[
            jax.ShapeDtypeStruct((B, N * K, 1), jnp.float32),
            jax.ShapeDtypeStruct((B, N, D_OBJ), jnp.float32),
            jax.ShapeDtypeStruct((B, N, 1), jnp.float32),
        ],
    )(obj, g1, ub, maskf, tabg, dk, sp_w3, sp_b2, pr_w3, pr_b2, sc_w,
      vl_w, vl_b2, fu_w1a, langc, fu_b12, fu_w2, fu_b22, cl_w, cl_b2)


def _lang_body(txt_ref, le_w1_ref, le_b1_ref, le_w2_ref, le_b2_ref,
               fu_w1b_ref, lang_ref, langc_ref):
    h = jnp.maximum(jnp.dot(txt_ref[...], le_w1_ref[...],
                            preferred_element_type=jnp.float32)
                    + le_b1_ref[...], 0.0)
    lang = jnp.dot(h, le_w2_ref[...],
                   preferred_element_type=jnp.float32) + le_b2_ref[...]
    lang_ref[...] = lang
    langc_ref[...] = jnp.dot(lang, fu_w1b_ref[...],
                             preferred_element_type=jnp.float32)


def _lang(text_features, le_w1, le_b12, le_w2, le_b22, fu_w1b):
    return pl.pallas_call(
        _lang_body,
        out_shape=[
            jax.ShapeDtypeStruct((B, D_LANG), jnp.float32),
            jax.ShapeDtypeStruct((B, D_FUS), jnp.float32),
        ],
    )(text_features, le_w1, le_b12, le_w2, le_b22, fu_w1b)


def kernel(points, object_mask, text_features, centers, sizes, pe_w1, pe_b1,
           pe_w2, pe_b2, le_w1, le_b1, le_w2, le_b2, sp_w, sp_b, pr_w, pr_b,
           sc_w, sc_b, vl_w, vl_b, fu_w1, fu_b1, fu_w2, fu_b2, cl_w, cl_b):
    maskf = object_mask.astype(jnp.float32)[:, :, None]
    maskf_row = object_mask.astype(jnp.float32)[:, None, :]
    centers_t = jnp.transpose(centers, (0, 2, 1))

    obj, g1, tab, ub, nbr, nbrg, dval = _encode(
        points, centers, sizes, maskf, centers_t, maskf_row,
        pe_w1, pe_b1[None, :], pe_w2, pe_b2[None, :],
        pr_w[0:D_OBJ], pr_w[D_OBJ:2 * D_OBJ], sp_w)

    tabg = _sc_gather(nbrg.reshape(-1), tab.reshape(B * N, D_RH))

    lang, langc = _lang(text_features, le_w1, le_b1[None, :], le_w2,
                        le_b2[None, :], fu_w1[D_OBJ:])

    wgt2, enhanced, logits3 = _relation(
        obj, g1, ub, maskf, tabg.reshape(B, N * K, D_RH),
        dval.reshape(B, N * K, 1), sp_w[3:4], sp_b[None, :],
        pr_w[2 * D_OBJ:], pr_b[None, :], sc_w, vl_w, vl_b[None, :],
        fu_w1[0:D_OBJ], langc[:, None, :], fu_b1[None, :], fu_w2,
        fu_b2[None, :],
        cl_w, cl_b[None, :])

    wgt = wgt2.reshape(B, N, K)
    logits = logits3[:, :, 0]
    return (logits, enhanced, obj, lang, wgt, nbr)


# speedup vs baseline: 1.2269x; 1.1160x over previous
import functools

import jax
import jax.numpy as jnp
from jax import lax
from jax.experimental import pallas as pl
from jax.experimental.pallas import tpu as pltpu
from jax.experimental.pallas import tpu_sc as plsc

B, N, K = 8, 1024, 8
D_PT, D_PH, D_OBJ = 256, 128, 256
D_LIN, D_LH, D_LANG = 768, 256, 256
D_RH = 128
D_FUS = 512
DIAM = 5.0

TA = 512
TN = 256

SC_NC, SC_NS = 2, 16
SC_NW = SC_NC * SC_NS
SC_ROWS = B * N * K
SC_RPW = SC_ROWS // SC_NW
SC_CH = 128
SC_NCH = SC_RPW // SC_CH


def _enc_body(pts_ref, cen_ref, siz_ref, mf_ref, cent_ref, mrow_ref,
              pe_w1_ref, pe_b1_ref, pe_w2_ref, pe_b2_ref,
              pr_w1_ref, pr_w2_ref, sp_w_ref,
              txt_ref, le_w1_ref, le_b1_ref, le_w2_ref, le_b2_ref,
              fu_w1b_ref,
              obj_ref, g1_ref, tab_ref, ub_ref,
              nbr_ref, nbrg_ref, dval_ref, lang_ref, langc_ref):
    @pl.when(jnp.logical_and(pl.program_id(0) == 0, pl.program_id(1) == 0))
    def _():
        hl = jnp.maximum(jnp.dot(txt_ref[...], le_w1_ref[...],
                                 preferred_element_type=jnp.float32)
                         + le_b1_ref[...], 0.0)
        lang = jnp.dot(hl, le_w2_ref[...],
                       preferred_element_type=jnp.float32) + le_b2_ref[...]
        lang_ref[...] = lang
        langc_ref[...] = jnp.dot(lang, fu_w1b_ref[...],
                                 preferred_element_type=jnp.float32)
    b = pl.program_id(0)
    i = pl.program_id(1)
    cb = cen_ref[0]
    ct = cent_ref[0]
    dx = cb[:, 0:1] - ct[0:1, :]
    dy = cb[:, 1:2] - ct[1:2, :]
    dz = cb[:, 2:3] - ct[2:3, :]
    dist = jnp.sqrt(dx * dx + dy * dy + dz * dz + 1e-12) / DIAM
    col = lax.broadcasted_iota(jnp.int32, (TA, N), 1)
    row = i * TA + lax.broadcasted_iota(jnp.int32, (TA, N), 0)
    valid = (mrow_ref[0] > 0.0) & (col != row)
    dm = jnp.where(valid, dist, jnp.inf)
    for k in range(K):
        minv = jnp.min(dm, axis=1, keepdims=True)
        idx = jnp.min(jnp.where(dm == minv, col, N), axis=1, keepdims=True)
        nbr_ref[0, :, k:k + 1] = idx
        nbrg_ref[0, :, k:k + 1] = idx + b * N
        dval_ref[0, :, k:k + 1] = minv
        dm = jnp.where(col == idx, jnp.inf, dm)

    pts = pts_ref[0]
    h = jnp.maximum(jnp.dot(pts, pe_w1_ref[...],
                            preferred_element_type=jnp.float32)
                    + pe_b1_ref[...], 0.0)
    obj = (jnp.dot(h, pe_w2_ref[...], preferred_element_type=jnp.float32)
           + pe_b2_ref[...]) * mf_ref[0]
    obj_ref[0] = obj
    g1_ref[0] = jnp.dot(obj, pr_w1_ref[...], preferred_element_type=jnp.float32)
    cw = jnp.dot(cen_ref[0], sp_w_ref[0:3, :],
                 preferred_element_type=jnp.float32) * (1.0 / DIAM)
    g2 = jnp.dot(obj, pr_w2_ref[...], preferred_element_type=jnp.float32)
    ua = cw + jnp.dot(siz_ref[0], sp_w_ref[7:10, :],
                      preferred_element_type=jnp.float32)
    g2i = lax.bitcast_convert_type(g2, jnp.int32)
    uai = lax.bitcast_convert_type(ua, jnp.int32)
    hi = (g2i + 0x8000) & jnp.int32(-65536)
    lo = jnp.right_shift(uai + 0x8000, 16) & 0xFFFF
    tab_ref[0] = hi | lo
    ub_ref[0] = jnp.dot(siz_ref[0], sp_w_ref[4:7, :],
                        preferred_element_type=jnp.float32) - cw


def _encode(points, centers, sizes, maskf, centers_t, maskf_row,
            pe_w1, pe_b1, pe_w2, pe_b2, pr_w1, pr_w2, sp_w,
            text_features, le_w1, le_b1, le_w2, le_b2, fu_w1b):
    grid = (B, N // TA)
    return pl.pallas_call(
        _enc_body,
        grid=grid,
        in_specs=[
            pl.BlockSpec((1, TA, D_PT), lambda b, i: (b, i, 0)),
            pl.BlockSpec((1, TA, 3), lambda b, i: (b, i, 0)),
            pl.BlockSpec((1, TA, 3), lambda b, i: (b, i, 0)),
            pl.BlockSpec((1, TA, 1), lambda b, i: (b, i, 0)),
            pl.BlockSpec((1, 3, N), lambda b, i: (b, 0, 0)),
            pl.BlockSpec((1, 1, N), lambda b, i: (b, 0, 0)),
            pl.BlockSpec((D_PT, D_PH), lambda b, i: (0, 0)),
            pl.BlockSpec((1, D_PH), lambda b, i: (0, 0)),
            pl.BlockSpec((D_PH, D_OBJ), lambda b, i: (0, 0)),
            pl.BlockSpec((1, D_OBJ), lambda b, i: (0, 0)),
            pl.BlockSpec((D_OBJ, D_RH), lambda b, i: (0, 0)),
            pl.BlockSpec((D_OBJ, D_RH), lambda b, i: (0, 0)),
            pl.BlockSpec((10, D_RH), lambda b, i: (0, 0)),
            pl.BlockSpec((B, D_LIN), lambda b, i: (0, 0)),
            pl.BlockSpec((D_LIN, D_LH), lambda b, i: (0, 0)),
            pl.BlockSpec((1, D_LH), lambda b, i: (0, 0)),
            pl.BlockSpec((D_LH, D_LANG), lambda b, i: (0, 0)),
            pl.BlockSpec((1, D_LANG), lambda b, i: (0, 0)),
            pl.BlockSpec((D_LANG, D_FUS), lambda b, i: (0, 0)),
        ],
        out_specs=[
            pl.BlockSpec((1, TA, D_OBJ), lambda b, i: (b, i, 0)),
            pl.BlockSpec((1, TA, D_RH), lambda b, i: (b, i, 0)),
            pl.BlockSpec((1, TA, D_RH), lambda b, i: (b, i, 0)),
            pl.BlockSpec((1, TA, D_RH), lambda b, i: (b, i, 0)),
            pl.BlockSpec((1, TA, K), lambda b, i: (b, i, 0)),
            pl.BlockSpec((1, TA, K), lambda b, i: (b, i, 0)),
            pl.BlockSpec((1, TA, K), lambda b, i: (b, i, 0)),
            pl.BlockSpec((B, D_LANG), lambda b, i: (0, 0)),
            pl.BlockSpec((B, D_FUS), lambda b, i: (0, 0)),
        ],
        out_shape=[
            jax.ShapeDtypeStruct((B, N, D_OBJ), jnp.float32),
            jax.ShapeDtypeStruct((B, N, D_RH), jnp.float32),
            jax.ShapeDtypeStruct((B, N, D_RH), jnp.int32),
            jax.ShapeDtypeStruct((B, N, D_RH), jnp.float32),
            jax.ShapeDtypeStruct((B, N, K), jnp.int32),
            jax.ShapeDtypeStruct((B, N, K), jnp.int32),
            jax.ShapeDtypeStruct((B, N, K), jnp.float32),
            jax.ShapeDtypeStruct((B, D_LANG), jnp.float32),
            jax.ShapeDtypeStruct((B, D_FUS), jnp.float32),
        ],
    )(points, centers, sizes, maskf, centers_t, maskf_row,
      pe_w1, pe_b1, pe_w2, pe_b2, pr_w1, pr_w2, sp_w,
      text_features, le_w1, le_b1, le_w2, le_b2, fu_w1b)


def _sc_gather_body(idx_hbm, tab_hbm, out_hbm, idx_v,
                    buf0, buf1, gs0, gs1, ws0, ws1):
    wid = lax.axis_index("s") * SC_NC + lax.axis_index("c")
    base = wid * SC_RPW
    pltpu.sync_copy(idx_hbm.at[pl.ds(base, SC_RPW)], idx_v)
    bufs = (buf0, buf1)
    gsem = (gs0, gs1)
    wsem = (ws0, ws1)
    gops = [None, None]
    wops = [None, None]
    for ci in range(SC_NCH):
        s = ci % 2
        if wops[s] is not None:
            wops[s].wait()
        gops[s] = pltpu.async_copy(
            tab_hbm.at[idx_v.at[pl.ds(ci * SC_CH, SC_CH)]], bufs[s], gsem[s])
        p = 1 - s
        if gops[p] is not None:
            gops[p].wait()
            wops[p] = pltpu.async_copy(
                bufs[p], out_hbm.at[pl.ds(base + (ci - 1) * SC_CH, SC_CH)],
                wsem[p])
    last = (SC_NCH - 1) % 2
    gops[last].wait()
    wops[last] = pltpu.async_copy(
        bufs[last], out_hbm.at[pl.ds(base + (SC_NCH - 1) * SC_CH, SC_CH)],
        wsem[last])
    wops[1 - last].wait()
    wops[last].wait()


@functools.cache
def _sc_gather_kernel():
    return pl.kernel(
        _sc_gather_body,
        out_type=jax.ShapeDtypeStruct((SC_ROWS, D_RH), jnp.int32),
        mesh=plsc.VectorSubcoreMesh(core_axis_name="c", subcore_axis_name="s",
                                    num_cores=SC_NC, num_subcores=SC_NS),
        scratch_types=[
            pltpu.VMEM((SC_RPW,), jnp.int32),
            pltpu.VMEM((SC_CH, D_RH), jnp.int32),
            pltpu.VMEM((SC_CH, D_RH), jnp.int32),
            pltpu.SemaphoreType.DMA,
            pltpu.SemaphoreType.DMA,
            pltpu.SemaphoreType.DMA,
            pltpu.SemaphoreType.DMA,
        ],
    )


def _sc_gather(nbrg_flat, tab):
    return _sc_gather_kernel()(nbrg_flat, tab)


def _rel_body(obj_ref, g1_ref, ub_ref, mf_ref, tabg_ref, dk_ref,
              sp_w3_ref, sp_b_ref, pr_w3_ref, pr_b_ref, sc_w_ref,
              vl_w_ref, vl_b_ref, fu_w1a_ref, langc_ref, fu_b1_ref,
              fu_w2_ref, fu_b2_ref, cl_w_ref, cl_b_ref,
              wgt_ref, enh_ref, log_ref):
    tabg3 = tabg_ref[0].reshape(TN, K, D_RH)
    g2j3 = lax.bitcast_convert_type(tabg3 & jnp.int32(-65536), jnp.float32)
    uaj3 = lax.bitcast_convert_type(jnp.left_shift(tabg3, 16), jnp.float32)
    hdk = jnp.dot(dk_ref[0], sp_w3_ref[...],
                  preferred_element_type=jnp.float32)
    hsp3 = jnp.maximum(
        uaj3
        + ub_ref[0].reshape(TN, 1, D_RH)
        + hdk.reshape(TN, K, D_RH)
        + sp_b_ref[...].reshape(1, 1, D_RH), 0.0)

    hp = jnp.dot(hsp3.reshape(TN * K, D_RH), pr_w3_ref[...],
                 preferred_element_type=jnp.float32)
    pair3 = jnp.maximum(
        g1_ref[0].reshape(TN, 1, D_RH)
        + g2j3
        + hp.reshape(TN, K, D_RH)
        + pr_b_ref[...].reshape(1, 1, D_RH), 0.0)
    pair2 = pair3.reshape(TN * K, D_RH)

    w_rep = jnp.broadcast_to(sc_w_ref[0:D_RH, :], (D_RH, D_RH))
    score = jnp.dot(pair2, w_rep,
                    preferred_element_type=jnp.float32).reshape(TN, K, D_RH)
    m = jnp.max(score, axis=1, keepdims=True)
    e = jnp.exp(score - m)
    s = jnp.sum(e, axis=1, keepdims=True)

    score_c = jnp.dot(pair2, sc_w_ref[0:D_RH, :],
                      preferred_element_type=jnp.float32).reshape(TN, K, 1)
    mc = jnp.max(score_c, axis=1, keepdims=True)
    ec = jnp.exp(score_c - mc)
    sc = jnp.sum(ec, axis=1, keepdims=True)
    wgt_ref[0] = (ec / sc).reshape(TN * K, 1)

    pbar = jnp.sum(e * pair3, axis=1) / s[:, 0, :]
    ctx = jnp.dot(pbar, vl_w_ref[...],
                  preferred_element_type=jnp.float32) + vl_b_ref[...]
    mf = mf_ref[0]
    enh = (obj_ref[0] + ctx) * mf
    enh_ref[0] = enh

    f1 = jnp.maximum(jnp.dot(enh, fu_w1a_ref[...],
                             preferred_element_type=jnp.float32)
                     + langc_ref[0] + fu_b1_ref[...], 0.0)
    f2 = jnp.maximum(jnp.dot(f1, fu_w2_ref[...],
                             preferred_element_type=jnp.float32)
                     + fu_b2_ref[...], 0.0)
    logit = jnp.dot(f2, cl_w_ref[...],
                    preferred_element_type=jnp.float32) + cl_b_ref[...]
    log_ref[0] = jnp.where(mf > 0.0, logit, -jnp.inf)


def _relation(obj, g1, ub, maskf, tabg, dk, sp_w3, sp_b2, pr_w3, pr_b2,
              sc_w, vl_w, vl_b2, fu_w1a, langc, fu_b12, fu_w2, fu_b22,
              cl_w, cl_b2):
    grid = (B, N // TN)
    return pl.pallas_call(
        _rel_body,
        grid=grid,
        in_specs=[
            pl.BlockSpec((1, TN, D_OBJ), lambda b, i: (b, i, 0)),
            pl.BlockSpec((1, TN, D_RH), lambda b, i: (b, i, 0)),
            pl.BlockSpec((1, TN, D_RH), lambda b, i: (b, i, 0)),
            pl.BlockSpec((1, TN, 1), lambda b, i: (b, i, 0)),
            pl.BlockSpec((1, TN * K, D_RH), lambda b, i: (b, i, 0)),
            pl.BlockSpec((1, TN * K, 1), lambda b, i: (b, i, 0)),
            pl.BlockSpec((1, D_RH), lambda b, i: (0, 0)),
            pl.BlockSpec((1, D_RH), lambda b, i: (0, 0)),
            pl.BlockSpec((D_RH, D_RH), lambda b, i: (0, 0)),
            pl.BlockSpec((1, D_RH), lambda b, i: (0, 0)),
            pl.BlockSpec((D_RH + D_LANG, 1), lambda b, i: (0, 0)),
            pl.BlockSpec((D_RH, D_OBJ), lambda b, i: (0, 0)),
            pl.BlockSpec((1, D_OBJ), lambda b, i: (0, 0)),
            pl.BlockSpec((D_OBJ, D_FUS), lambda b, i: (0, 0)),
            pl.BlockSpec((1, 1, D_FUS), lambda b, i: (b, 0, 0)),
            pl.BlockSpec((1, D_FUS), lambda b, i: (0, 0)),
            pl.BlockSpec((D_FUS, D_FUS), lambda b, i: (0, 0)),
            pl.BlockSpec((1, D_FUS), lambda b, i: (0, 0)),
            pl.BlockSpec((D_FUS, 1), lambda b, i: (0, 0)),
            pl.BlockSpec((1, 1), lambda b, i: (0, 0)),
        ],
        out_specs=[
            pl.BlockSpec((1, TN * K, 1), lambda b, i: (b, i, 0)),
            pl.BlockSpec((1, TN, D_OBJ), lambda b, i: (b, i, 0)),
            pl.BlockSpec((1, TN, 1), lambda b, i: (b, i, 0)),
        ],
        out_shape=[
            jax.ShapeDtypeStruct((B, N * K, 1), jnp.float32),
            jax.ShapeDtypeStruct((B, N, D_OBJ), jnp.float32),
            jax.ShapeDtypeStruct((B, N, 1), jnp.float32),
        ],
    )(obj, g1, ub, maskf, tabg, dk, sp_w3, sp_b2, pr_w3, pr_b2, sc_w,
      vl_w, vl_b2, fu_w1a, langc, fu_b12, fu_w2, fu_b22, cl_w, cl_b2)


def kernel(points, object_mask, text_features, centers, sizes, pe_w1, pe_b1,
           pe_w2, pe_b2, le_w1, le_b1, le_w2, le_b2, sp_w, sp_b, pr_w, pr_b,
           sc_w, sc_b, vl_w, vl_b, fu_w1, fu_b1, fu_w2, fu_b2, cl_w, cl_b):
    maskf = object_mask.astype(jnp.float32)[:, :, None]
    maskf_row = object_mask.astype(jnp.float32)[:, None, :]
    centers_t = jnp.transpose(centers, (0, 2, 1))

    obj, g1, tab, ub, nbr, nbrg, dval, lang, langc = _encode(
        points, centers, sizes, maskf, centers_t, maskf_row,
        pe_w1, pe_b1[None, :], pe_w2, pe_b2[None, :],
        pr_w[0:D_OBJ], pr_w[D_OBJ:2 * D_OBJ], sp_w,
        text_features, le_w1, le_b1[None, :], le_w2, le_b2[None, :],
        fu_w1[D_OBJ:])

    tabg = _sc_gather(nbrg.reshape(-1), tab.reshape(B * N, D_RH))

    wgt2, enhanced, logits3 = _relation(
        obj, g1, ub, maskf, tabg.reshape(B, N * K, D_RH),
        dval.reshape(B, N * K, 1), sp_w[3:4], sp_b[None, :],
        pr_w[2 * D_OBJ:], pr_b[None, :], sc_w, vl_w, vl_b[None, :],
        fu_w1[0:D_OBJ], langc[:, None, :], fu_b1[None, :], fu_w2,
        fu_b2[None, :],
        cl_w, cl_b[None, :])

    wgt = wgt2.reshape(B, N, K)
    logits = logits3[:, :, 0]
    return (logits, enhanced, obj, lang, wgt, nbr)


# speedup vs baseline: 1.3077x; 1.0658x over previous
import functools

import jax
import jax.numpy as jnp
from jax import lax
from jax.experimental import pallas as pl
from jax.experimental.pallas import tpu as pltpu
from jax.experimental.pallas import tpu_sc as plsc

B, N, K = 8, 1024, 8
D_PT, D_PH, D_OBJ = 256, 128, 256
D_LIN, D_LH, D_LANG = 768, 256, 256
D_RH = 128
D_FUS = 512
DIAM = 5.0

TA = 512
TN = 256

SC_NC, SC_NS = 2, 16
SC_NW = SC_NC * SC_NS
SC_ROWS = B * N * K
SC_RPW = SC_ROWS // SC_NW
SC_CH = 128
SC_NCH = SC_RPW // SC_CH


def _enc_body(pts_ref, cen_ref, siz_ref, mf_ref, cent_ref, mrow_ref,
              pe_w1_ref, pe_b1_ref, pe_w2_ref, pe_b2_ref,
              pr_w1_ref, pr_w2_ref, sp_w_ref,
              txt_ref, le_w1_ref, le_b1_ref, le_w2_ref, le_b2_ref,
              fu_w1b_ref,
              obj_ref, g1_ref, tab_ref, ub_ref,
              nbr_ref, nbrg_ref, dval_ref, lang_ref, langc_ref):
    @pl.when(jnp.logical_and(pl.program_id(0) == 0, pl.program_id(1) == 0))
    def _():
        hl = jnp.maximum(jnp.dot(txt_ref[...], le_w1_ref[...],
                                 preferred_element_type=jnp.float32)
                         + le_b1_ref[...], 0.0)
        lang = jnp.dot(hl, le_w2_ref[...],
                       preferred_element_type=jnp.float32) + le_b2_ref[...]
        lang_ref[...] = lang
        langc_ref[...] = jnp.dot(lang, fu_w1b_ref[...],
                                 preferred_element_type=jnp.float32)
    b = pl.program_id(0)
    i = pl.program_id(1)
    cb = cen_ref[0]
    ct = cent_ref[0]
    dx = cb[:, 0:1] - ct[0:1, :]
    dy = cb[:, 1:2] - ct[1:2, :]
    dz = cb[:, 2:3] - ct[2:3, :]
    dist = jnp.sqrt(dx * dx + dy * dy + dz * dz + 1e-12) / DIAM
    col = lax.broadcasted_iota(jnp.int32, (TA, N), 1)
    row = i * TA + lax.broadcasted_iota(jnp.int32, (TA, N), 0)
    valid = (mrow_ref[0] > 0.0) & (col != row)
    dm = jnp.where(valid, dist, jnp.inf)
    for k in range(K):
        minv = jnp.min(dm, axis=1, keepdims=True)
        idx = jnp.min(jnp.where(dm == minv, col, N), axis=1, keepdims=True)
        nbr_ref[0, :, k:k + 1] = idx
        nbrg_ref[0, :, k:k + 1] = idx + b * N
        dval_ref[0, :, k:k + 1] = minv
        dm = jnp.where(col == idx, jnp.inf, dm)

    pts = pts_ref[0]
    h = jnp.maximum(jnp.dot(pts, pe_w1_ref[...],
                            preferred_element_type=jnp.float32)
                    + pe_b1_ref[...], 0.0)
    obj = (jnp.dot(h, pe_w2_ref[...], preferred_element_type=jnp.float32)
           + pe_b2_ref[...]) * mf_ref[0]
    obj_ref[0] = obj
    g1_ref[0] = jnp.dot(obj, pr_w1_ref[...], preferred_element_type=jnp.float32)
    cw = jnp.dot(cen_ref[0], sp_w_ref[0:3, :],
                 preferred_element_type=jnp.float32) * (1.0 / DIAM)
    g2 = jnp.dot(obj, pr_w2_ref[...], preferred_element_type=jnp.float32)
    ua = cw + jnp.dot(siz_ref[0], sp_w_ref[7:10, :],
                      preferred_element_type=jnp.float32)
    g2i = lax.bitcast_convert_type(g2, jnp.int32)
    uai = lax.bitcast_convert_type(ua, jnp.int32)
    hi = (g2i + 0x8000) & jnp.int32(-65536)
    lo = jnp.right_shift(uai + 0x8000, 16) & 0xFFFF
    tab_ref[0] = hi | lo
    ub_ref[0] = jnp.dot(siz_ref[0], sp_w_ref[4:7, :],
                        preferred_element_type=jnp.float32) - cw


def _encode(points, centers, sizes, maskf, centers_t, maskf_row,
            pe_w1, pe_b1, pe_w2, pe_b2, pr_w1, pr_w2, sp_w,
            text_features, le_w1, le_b1, le_w2, le_b2, fu_w1b):
    grid = (B, N // TA)
    return pl.pallas_call(
        _enc_body,
        grid=grid,
        in_specs=[
            pl.BlockSpec((1, TA, D_PT), lambda b, i: (b, i, 0)),
            pl.BlockSpec((1, TA, 3), lambda b, i: (b, i, 0)),
            pl.BlockSpec((1, TA, 3), lambda b, i: (b, i, 0)),
            pl.BlockSpec((1, TA, 1), lambda b, i: (b, i, 0)),
            pl.BlockSpec((1, 3, N), lambda b, i: (b, 0, 0)),
            pl.BlockSpec((1, 1, N), lambda b, i: (b, 0, 0)),
            pl.BlockSpec((D_PT, D_PH), lambda b, i: (0, 0)),
            pl.BlockSpec((1, D_PH), lambda b, i: (0, 0)),
            pl.BlockSpec((D_PH, D_OBJ), lambda b, i: (0, 0)),
            pl.BlockSpec((1, D_OBJ), lambda b, i: (0, 0)),
            pl.BlockSpec((D_OBJ, D_RH), lambda b, i: (0, 0)),
            pl.BlockSpec((D_OBJ, D_RH), lambda b, i: (0, 0)),
            pl.BlockSpec((10, D_RH), lambda b, i: (0, 0)),
            pl.BlockSpec((B, D_LIN), lambda b, i: (0, 0)),
            pl.BlockSpec((D_LIN, D_LH), lambda b, i: (0, 0)),
            pl.BlockSpec((1, D_LH), lambda b, i: (0, 0)),
            pl.BlockSpec((D_LH, D_LANG), lambda b, i: (0, 0)),
            pl.BlockSpec((1, D_LANG), lambda b, i: (0, 0)),
            pl.BlockSpec((D_LANG, D_FUS), lambda b, i: (0, 0)),
        ],
        out_specs=[
            pl.BlockSpec((1, TA, D_OBJ), lambda b, i: (b, i, 0)),
            pl.BlockSpec((1, TA, D_RH), lambda b, i: (b, i, 0)),
            pl.BlockSpec((1, TA, D_RH), lambda b, i: (b, i, 0)),
            pl.BlockSpec((1, TA, D_RH), lambda b, i: (b, i, 0)),
            pl.BlockSpec((1, TA, K), lambda b, i: (b, i, 0)),
            pl.BlockSpec((1, TA, K), lambda b, i: (b, i, 0)),
            pl.BlockSpec((1, TA, K), lambda b, i: (b, i, 0)),
            pl.BlockSpec((B, D_LANG), lambda b, i: (0, 0)),
            pl.BlockSpec((B, D_FUS), lambda b, i: (0, 0)),
        ],
        out_shape=[
            jax.ShapeDtypeStruct((B, N, D_OBJ), jnp.float32),
            jax.ShapeDtypeStruct((B, N, D_RH), jnp.float32),
            jax.ShapeDtypeStruct((B, N, D_RH), jnp.int32),
            jax.ShapeDtypeStruct((B, N, D_RH), jnp.float32),
            jax.ShapeDtypeStruct((B, N, K), jnp.int32),
            jax.ShapeDtypeStruct((B, N, K), jnp.int32),
            jax.ShapeDtypeStruct((B, N, K), jnp.float32),
            jax.ShapeDtypeStruct((B, D_LANG), jnp.float32),
            jax.ShapeDtypeStruct((B, D_FUS), jnp.float32),
        ],
    )(points, centers, sizes, maskf, centers_t, maskf_row,
      pe_w1, pe_b1, pe_w2, pe_b2, pr_w1, pr_w2, sp_w,
      text_features, le_w1, le_b1, le_w2, le_b2, fu_w1b)


def _sc_gather_body(idx_hbm, tab_hbm, out_hbm, idx_v,
                    buf0, buf1, gs0, gs1, ws0, ws1):
    wid = lax.axis_index("s") * SC_NC + lax.axis_index("c")
    base = wid * SC_RPW
    pltpu.sync_copy(idx_hbm.at[pl.ds(base, SC_RPW)], idx_v)
    bufs = (buf0, buf1)
    gsem = (gs0, gs1)
    wsem = (ws0, ws1)
    gops = [None, None]
    wops = [None, None]
    for ci in range(SC_NCH):
        s = ci % 2
        if wops[s] is not None:
            wops[s].wait()
        gops[s] = pltpu.async_copy(
            tab_hbm.at[idx_v.at[pl.ds(ci * SC_CH, SC_CH)]], bufs[s], gsem[s])
        p = 1 - s
        if gops[p] is not None:
            gops[p].wait()
            wops[p] = pltpu.async_copy(
                bufs[p], out_hbm.at[pl.ds(base + (ci - 1) * SC_CH, SC_CH)],
                wsem[p])
    last = (SC_NCH - 1) % 2
    gops[last].wait()
    wops[last] = pltpu.async_copy(
        bufs[last], out_hbm.at[pl.ds(base + (SC_NCH - 1) * SC_CH, SC_CH)],
        wsem[last])
    wops[1 - last].wait()
    wops[last].wait()


@functools.cache
def _sc_gather_kernel():
    return pl.kernel(
        _sc_gather_body,
        out_type=jax.ShapeDtypeStruct((SC_ROWS, D_RH), jnp.int32),
        mesh=plsc.VectorSubcoreMesh(core_axis_name="c", subcore_axis_name="s",
                                    num_cores=SC_NC, num_subcores=SC_NS),
        scratch_types=[
            pltpu.VMEM((SC_RPW,), jnp.int32),
            pltpu.VMEM((SC_CH, D_RH), jnp.int32),
            pltpu.VMEM((SC_CH, D_RH), jnp.int32),
            pltpu.SemaphoreType.DMA,
            pltpu.SemaphoreType.DMA,
            pltpu.SemaphoreType.DMA,
            pltpu.SemaphoreType.DMA,
        ],
    )


def _sc_gather(nbrg_flat, tab):
    return _sc_gather_kernel()(nbrg_flat, tab)


def _rel_body(obj_ref, g1_ref, ub_ref, mf_ref, tabg_ref, dk_ref,
              sp_w3_ref, sp_b_ref, pr_w3_ref, pr_b_ref, sc_w_ref,
              vl_w_ref, vl_b_ref, fu_w1a_ref, langc_ref, fu_b1_ref,
              fu_w2_ref, fu_b2_ref, cl_w_ref, cl_b_ref,
              wgt_ref, enh_ref, log_ref):
    tabg3 = tabg_ref[0].reshape(TN, K, D_RH)
    g2j3 = lax.bitcast_convert_type(tabg3 & jnp.int32(-65536), jnp.float32)
    uaj3 = lax.bitcast_convert_type(jnp.left_shift(tabg3, 16), jnp.float32)
    hdk = jnp.dot(dk_ref[0], sp_w3_ref[...],
                  preferred_element_type=jnp.float32)
    hsp3 = jnp.maximum(
        uaj3
        + ub_ref[0].reshape(TN, 1, D_RH)
        + hdk.reshape(TN, K, D_RH)
        + sp_b_ref[...].reshape(1, 1, D_RH), 0.0)

    hp = jnp.dot(hsp3.reshape(TN * K, D_RH), pr_w3_ref[...],
                 preferred_element_type=jnp.float32)
    pair3 = jnp.maximum(
        g1_ref[0].reshape(TN, 1, D_RH)
        + g2j3
        + hp.reshape(TN, K, D_RH)
        + pr_b_ref[...].reshape(1, 1, D_RH), 0.0)
    pair2 = pair3.reshape(TN * K, D_RH)

    score_c = jnp.dot(pair2, sc_w_ref[0:D_RH, :],
                      preferred_element_type=jnp.float32).reshape(TN, K, 1)
    mc = jnp.max(score_c, axis=1, keepdims=True)
    ec = jnp.exp(score_c - mc)
    sc = jnp.sum(ec, axis=1, keepdims=True)
    wgtc = ec / sc
    wgt_ref[0] = wgtc.reshape(TN * K, 1)

    wgt_rep = jnp.dot(wgtc.reshape(TN * K, 1), jnp.ones((1, D_RH), jnp.float32),
                      preferred_element_type=jnp.float32)
    pbar = jnp.sum(wgt_rep.reshape(TN, K, D_RH) * pair3, axis=1)
    ctx = jnp.dot(pbar, vl_w_ref[...],
                  preferred_element_type=jnp.float32) + vl_b_ref[...]
    mf = mf_ref[0]
    enh = (obj_ref[0] + ctx) * mf
    enh_ref[0] = enh

    f1 = jnp.maximum(jnp.dot(enh, fu_w1a_ref[...],
                             preferred_element_type=jnp.float32)
                     + langc_ref[0] + fu_b1_ref[...], 0.0)
    f2 = jnp.maximum(jnp.dot(f1, fu_w2_ref[...],
                             preferred_element_type=jnp.float32)
                     + fu_b2_ref[...], 0.0)
    logit = jnp.dot(f2, cl_w_ref[...],
                    preferred_element_type=jnp.float32) + cl_b_ref[...]
    log_ref[0] = jnp.where(mf > 0.0, logit, -jnp.inf)


def _relation(obj, g1, ub, maskf, tabg, dk, sp_w3, sp_b2, pr_w3, pr_b2,
              sc_w, vl_w, vl_b2, fu_w1a, langc, fu_b12, fu_w2, fu_b22,
              cl_w, cl_b2):
    grid = (B, N // TN)
    return pl.pallas_call(
        _rel_body,
        grid=grid,
        in_specs=[
            pl.BlockSpec((1, TN, D_OBJ), lambda b, i: (b, i, 0)),
            pl.BlockSpec((1, TN, D_RH), lambda b, i: (b, i, 0)),
            pl.BlockSpec((1, TN, D_RH), lambda b, i: (b, i, 0)),
            pl.BlockSpec((1, TN, 1), lambda b, i: (b, i, 0)),
            pl.BlockSpec((1, TN * K, D_RH), lambda b, i: (b, i, 0)),
            pl.BlockSpec((1, TN * K, 1), lambda b, i: (b, i, 0)),
            pl.BlockSpec((1, D_RH), lambda b, i: (0, 0)),
            pl.BlockSpec((1, D_RH), lambda b, i: (0, 0)),
            pl.BlockSpec((D_RH, D_RH), lambda b, i: (0, 0)),
            pl.BlockSpec((1, D_RH), lambda b, i: (0, 0)),
            pl.BlockSpec((D_RH + D_LANG, 1), lambda b, i: (0, 0)),
            pl.BlockSpec((D_RH, D_OBJ), lambda b, i: (0, 0)),
            pl.BlockSpec((1, D_OBJ), lambda b, i: (0, 0)),
            pl.BlockSpec((D_OBJ, D_FUS), lambda b, i: (0, 0)),
            pl.BlockSpec((1, 1, D_FUS), lambda b, i: (b, 0, 0)),
            pl.BlockSpec((1, D_FUS), lambda b, i: (0, 0)),
            pl.BlockSpec((D_FUS, D_FUS), lambda b, i: (0, 0)),
            pl.BlockSpec((1, D_FUS), lambda b, i: (0, 0)),
            pl.BlockSpec((D_FUS, 1), lambda b, i: (0, 0)),
            pl.BlockSpec((1, 1), lambda b, i: (0, 0)),
        ],
        out_specs=[
            pl.BlockSpec((1, TN * K, 1), lambda b, i: (b, i, 0)),
            pl.BlockSpec((1, TN, D_OBJ), lambda b, i: (b, i, 0)),
            pl.BlockSpec((1, TN, 1), lambda b, i: (b, i, 0)),
        ],
        out_shape=[
            jax.ShapeDtypeStruct((B, N * K, 1), jnp.float32),
            jax.ShapeDtypeStruct((B, N, D_OBJ), jnp.float32),
            jax.ShapeDtypeStruct((B, N, 1), jnp.float32),
        ],
    )(obj, g1, ub, maskf, tabg, dk, sp_w3, sp_b2, pr_w3, pr_b2, sc_w,
      vl_w, vl_b2, fu_w1a, langc, fu_b12, fu_w2, fu_b22, cl_w, cl_b2)


def kernel(points, object_mask, text_features, centers, sizes, pe_w1, pe_b1,
           pe_w2, pe_b2, le_w1, le_b1, le_w2, le_b2, sp_w, sp_b, pr_w, pr_b,
           sc_w, sc_b, vl_w, vl_b, fu_w1, fu_b1, fu_w2, fu_b2, cl_w, cl_b):
    maskf = object_mask.astype(jnp.float32)[:, :, None]
    maskf_row = object_mask.astype(jnp.float32)[:, None, :]
    centers_t = jnp.transpose(centers, (0, 2, 1))

    obj, g1, tab, ub, nbr, nbrg, dval, lang, langc = _encode(
        points, centers, sizes, maskf, centers_t, maskf_row,
        pe_w1, pe_b1[None, :], pe_w2, pe_b2[None, :],
        pr_w[0:D_OBJ], pr_w[D_OBJ:2 * D_OBJ], sp_w,
        text_features, le_w1, le_b1[None, :], le_w2, le_b2[None, :],
        fu_w1[D_OBJ:])

    tabg = _sc_gather(nbrg.reshape(-1), tab.reshape(B * N, D_RH))

    wgt2, enhanced, logits3 = _relation(
        obj, g1, ub, maskf, tabg.reshape(B, N * K, D_RH),
        dval.reshape(B, N * K, 1), sp_w[3:4], sp_b[None, :],
        pr_w[2 * D_OBJ:], pr_b[None, :], sc_w, vl_w, vl_b[None, :],
        fu_w1[0:D_OBJ], langc[:, None, :], fu_b1[None, :], fu_w2,
        fu_b2[None, :],
        cl_w, cl_b[None, :])

    wgt = wgt2.reshape(B, N, K)
    logits = logits3[:, :, 0]
    return (logits, enhanced, obj, lang, wgt, nbr)


# speedup vs baseline: 1.3782x; 1.0539x over previous
import functools

import jax
import jax.numpy as jnp
from jax import lax
from jax.experimental import pallas as pl
from jax.experimental.pallas import tpu as pltpu
from jax.experimental.pallas import tpu_sc as plsc

B, N, K = 8, 1024, 8
D_PT, D_PH, D_OBJ = 256, 128, 256
D_LIN, D_LH, D_LANG = 768, 256, 256
D_RH = 128
D_FUS = 512
DIAM = 5.0

TA = 1024
TN = 512

SC_NC, SC_NS = 2, 16
SC_NW = SC_NC * SC_NS
SC_ROWS = B * N * K
SC_RPW = SC_ROWS // SC_NW
SC_CH = 128
SC_NCH = SC_RPW // SC_CH


def _enc_body(pts_ref, cen_ref, siz_ref, mf_ref, cent_ref, mrow_ref,
              pe_w1_ref, pe_b1_ref, pe_w2_ref, pe_b2_ref,
              pr_w1_ref, pr_w2_ref, sp_w_ref,
              txt_ref, le_w1_ref, le_b1_ref, le_w2_ref, le_b2_ref,
              fu_w1b_ref,
              obj_ref, g1_ref, tab_ref, ub_ref,
              nbr_ref, nbrg_ref, dval_ref, lang_ref, langc_ref):
    @pl.when(jnp.logical_and(pl.program_id(0) == 0, pl.program_id(1) == 0))
    def _():
        hl = jnp.maximum(jnp.dot(txt_ref[...], le_w1_ref[...],
                                 preferred_element_type=jnp.float32)
                         + le_b1_ref[...], 0.0)
        lang = jnp.dot(hl, le_w2_ref[...],
                       preferred_element_type=jnp.float32) + le_b2_ref[...]
        lang_ref[...] = lang
        langc_ref[...] = jnp.dot(lang, fu_w1b_ref[...],
                                 preferred_element_type=jnp.float32)
    b = pl.program_id(0)
    i = pl.program_id(1)
    cb = cen_ref[0]
    ct = cent_ref[0]
    dx = cb[:, 0:1] - ct[0:1, :]
    dy = cb[:, 1:2] - ct[1:2, :]
    dz = cb[:, 2:3] - ct[2:3, :]
    sq = dx * dx + dy * dy + dz * dz
    col = lax.broadcasted_iota(jnp.int32, (TA, N), 1)
    row = i * TA + lax.broadcasted_iota(jnp.int32, (TA, N), 0)
    valid = (mrow_ref[0] > 0.0) & (col != row)
    dm = jnp.where(valid, sq, jnp.inf)
    for k in range(K):
        minv = jnp.min(dm, axis=1, keepdims=True)
        idx = jnp.min(jnp.where(dm == minv, col, N), axis=1, keepdims=True)
        nbr_ref[0, :, k:k + 1] = idx
        nbrg_ref[0, :, k:k + 1] = idx + b * N
        dval_ref[0, :, k:k + 1] = jnp.sqrt(minv + 1e-12) / DIAM
        dm = jnp.where(col == idx, jnp.inf, dm)

    pts = pts_ref[0]
    h = jnp.maximum(jnp.dot(pts, pe_w1_ref[...],
                            preferred_element_type=jnp.float32)
                    + pe_b1_ref[...], 0.0)
    obj = (jnp.dot(h, pe_w2_ref[...], preferred_element_type=jnp.float32)
           + pe_b2_ref[...]) * mf_ref[0]
    obj_ref[0] = obj
    g1_ref[0] = jnp.dot(obj, pr_w1_ref[...], preferred_element_type=jnp.float32)
    cw = jnp.dot(cen_ref[0], sp_w_ref[0:3, :],
                 preferred_element_type=jnp.float32) * (1.0 / DIAM)
    g2 = jnp.dot(obj, pr_w2_ref[...], preferred_element_type=jnp.float32)
    ua = cw + jnp.dot(siz_ref[0], sp_w_ref[7:10, :],
                      preferred_element_type=jnp.float32)
    g2i = lax.bitcast_convert_type(g2, jnp.int32)
    uai = lax.bitcast_convert_type(ua, jnp.int32)
    hi = (g2i + 0x8000) & jnp.int32(-65536)
    lo = jnp.right_shift(uai + 0x8000, 16) & 0xFFFF
    tab_ref[0] = hi | lo
    ub_ref[0] = jnp.dot(siz_ref[0], sp_w_ref[4:7, :],
                        preferred_element_type=jnp.float32) - cw


def _encode(points, centers, sizes, maskf, centers_t, maskf_row,
            pe_w1, pe_b1, pe_w2, pe_b2, pr_w1, pr_w2, sp_w,
            text_features, le_w1, le_b1, le_w2, le_b2, fu_w1b):
    grid = (B, N // TA)
    return pl.pallas_call(
        _enc_body,
        grid=grid,
        in_specs=[
            pl.BlockSpec((1, TA, D_PT), lambda b, i: (b, i, 0)),
            pl.BlockSpec((1, TA, 3), lambda b, i: (b, i, 0)),
            pl.BlockSpec((1, TA, 3), lambda b, i: (b, i, 0)),
            pl.BlockSpec((1, TA, 1), lambda b, i: (b, i, 0)),
            pl.BlockSpec((1, 3, N), lambda b, i: (b, 0, 0)),
            pl.BlockSpec((1, 1, N), lambda b, i: (b, 0, 0)),
            pl.BlockSpec((D_PT, D_PH), lambda b, i: (0, 0)),
            pl.BlockSpec((1, D_PH), lambda b, i: (0, 0)),
            pl.BlockSpec((D_PH, D_OBJ), lambda b, i: (0, 0)),
            pl.BlockSpec((1, D_OBJ), lambda b, i: (0, 0)),
            pl.BlockSpec((D_OBJ, D_RH), lambda b, i: (0, 0)),
            pl.BlockSpec((D_OBJ, D_RH), lambda b, i: (0, 0)),
            pl.BlockSpec((10, D_RH), lambda b, i: (0, 0)),
            pl.BlockSpec((B, D_LIN), lambda b, i: (0, 0)),
            pl.BlockSpec((D_LIN, D_LH), lambda b, i: (0, 0)),
            pl.BlockSpec((1, D_LH), lambda b, i: (0, 0)),
            pl.BlockSpec((D_LH, D_LANG), lambda b, i: (0, 0)),
            pl.BlockSpec((1, D_LANG), lambda b, i: (0, 0)),
            pl.BlockSpec((D_LANG, D_FUS), lambda b, i: (0, 0)),
        ],
        out_specs=[
            pl.BlockSpec((1, TA, D_OBJ), lambda b, i: (b, i, 0)),
            pl.BlockSpec((1, TA, D_RH), lambda b, i: (b, i, 0)),
            pl.BlockSpec((1, TA, D_RH), lambda b, i: (b, i, 0)),
            pl.BlockSpec((1, TA, D_RH), lambda b, i: (b, i, 0)),
            pl.BlockSpec((1, TA, K), lambda b, i: (b, i, 0)),
            pl.BlockSpec((1, TA, K), lambda b, i: (b, i, 0)),
            pl.BlockSpec((1, TA, K), lambda b, i: (b, i, 0)),
            pl.BlockSpec((B, D_LANG), lambda b, i: (0, 0)),
            pl.BlockSpec((B, D_FUS), lambda b, i: (0, 0)),
        ],
        out_shape=[
            jax.ShapeDtypeStruct((B, N, D_OBJ), jnp.float32),
            jax.ShapeDtypeStruct((B, N, D_RH), jnp.float32),
            jax.ShapeDtypeStruct((B, N, D_RH), jnp.int32),
            jax.ShapeDtypeStruct((B, N, D_RH), jnp.float32),
            jax.ShapeDtypeStruct((B, N, K), jnp.int32),
            jax.ShapeDtypeStruct((B, N, K), jnp.int32),
            jax.ShapeDtypeStruct((B, N, K), jnp.float32),
            jax.ShapeDtypeStruct((B, D_LANG), jnp.float32),
            jax.ShapeDtypeStruct((B, D_FUS), jnp.float32),
        ],
    )(points, centers, sizes, maskf, centers_t, maskf_row,
      pe_w1, pe_b1, pe_w2, pe_b2, pr_w1, pr_w2, sp_w,
      text_features, le_w1, le_b1, le_w2, le_b2, fu_w1b)


def _sc_gather_body(idx_hbm, tab_hbm, out_hbm, idx_v,
                    buf0, buf1, gs0, gs1, ws0, ws1):
    wid = lax.axis_index("s") * SC_NC + lax.axis_index("c")
    base = wid * SC_RPW
    pltpu.sync_copy(idx_hbm.at[pl.ds(base, SC_RPW)], idx_v)
    bufs = (buf0, buf1)
    gsem = (gs0, gs1)
    wsem = (ws0, ws1)
    gops = [None, None]
    wops = [None, None]
    for ci in range(SC_NCH):
        s = ci % 2
        if wops[s] is not None:
            wops[s].wait()
        gops[s] = pltpu.async_copy(
            tab_hbm.at[idx_v.at[pl.ds(ci * SC_CH, SC_CH)]], bufs[s], gsem[s])
        p = 1 - s
        if gops[p] is not None:
            gops[p].wait()
            wops[p] = pltpu.async_copy(
                bufs[p], out_hbm.at[pl.ds(base + (ci - 1) * SC_CH, SC_CH)],
                wsem[p])
    last = (SC_NCH - 1) % 2
    gops[last].wait()
    wops[last] = pltpu.async_copy(
        bufs[last], out_hbm.at[pl.ds(base + (SC_NCH - 1) * SC_CH, SC_CH)],
        wsem[last])
    wops[1 - last].wait()
    wops[last].wait()


@functools.cache
def _sc_gather_kernel():
    return pl.kernel(
        _sc_gather_body,
        out_type=jax.ShapeDtypeStruct((SC_ROWS, D_RH), jnp.int32),
        mesh=plsc.VectorSubcoreMesh(core_axis_name="c", subcore_axis_name="s",
                                    num_cores=SC_NC, num_subcores=SC_NS),
        scratch_types=[
            pltpu.VMEM((SC_RPW,), jnp.int32),
            pltpu.VMEM((SC_CH, D_RH), jnp.int32),
            pltpu.VMEM((SC_CH, D_RH), jnp.int32),
            pltpu.SemaphoreType.DMA,
            pltpu.SemaphoreType.DMA,
            pltpu.SemaphoreType.DMA,
            pltpu.SemaphoreType.DMA,
        ],
    )


def _sc_gather(nbrg_flat, tab):
    return _sc_gather_kernel()(nbrg_flat, tab)


def _rel_body(obj_ref, g1_ref, ub_ref, mf_ref, tabg_ref, dk_ref,
              sp_w3_ref, sp_b_ref, pr_w3_ref, pr_b_ref, sc_w_ref,
              vl_w_ref, vl_b_ref, fu_w1a_ref, langc_ref, fu_b1_ref,
              fu_w2_ref, fu_b2_ref, cl_w_ref, cl_b_ref,
              wgt_ref, enh_ref, log_ref):
    tabg3 = tabg_ref[0].reshape(TN, K, D_RH)
    g2j3 = lax.bitcast_convert_type(tabg3 & jnp.int32(-65536), jnp.float32)
    uaj3 = lax.bitcast_convert_type(jnp.left_shift(tabg3, 16), jnp.float32)
    hdk = jnp.dot(dk_ref[0], sp_w3_ref[...],
                  preferred_element_type=jnp.float32)
    hsp3 = jnp.maximum(
        uaj3
        + ub_ref[0].reshape(TN, 1, D_RH)
        + hdk.reshape(TN, K, D_RH)
        + sp_b_ref[...].reshape(1, 1, D_RH), 0.0)

    hp = jnp.dot(hsp3.reshape(TN * K, D_RH), pr_w3_ref[...],
                 preferred_element_type=jnp.float32)
    pair3 = jnp.maximum(
        g1_ref[0].reshape(TN, 1, D_RH)
        + g2j3
        + hp.reshape(TN, K, D_RH)
        + pr_b_ref[...].reshape(1, 1, D_RH), 0.0)
    pair2 = pair3.reshape(TN * K, D_RH)

    score_c = jnp.dot(pair2, sc_w_ref[0:D_RH, :],
                      preferred_element_type=jnp.float32).reshape(TN, K, 1)
    mc = jnp.max(score_c, axis=1, keepdims=True)
    ec = jnp.exp(score_c - mc)
    sc = jnp.sum(ec, axis=1, keepdims=True)
    wgtc = ec / sc
    wgt_ref[0] = wgtc.reshape(TN * K, 1)

    wgt_rep = jnp.dot(wgtc.reshape(TN * K, 1), jnp.ones((1, D_RH), jnp.float32),
                      preferred_element_type=jnp.float32)
    pbar = jnp.sum(wgt_rep.reshape(TN, K, D_RH) * pair3, axis=1)
    ctx = jnp.dot(pbar, vl_w_ref[...],
                  preferred_element_type=jnp.float32) + vl_b_ref[...]
    mf = mf_ref[0]
    enh = (obj_ref[0] + ctx) * mf
    enh_ref[0] = enh

    f1 = jnp.maximum(jnp.dot(enh, fu_w1a_ref[...],
                             preferred_element_type=jnp.float32)
                     + langc_ref[0] + fu_b1_ref[...], 0.0)
    f2 = jnp.maximum(jnp.dot(f1, fu_w2_ref[...],
                             preferred_element_type=jnp.float32)
                     + fu_b2_ref[...], 0.0)
    logit = jnp.dot(f2, cl_w_ref[...],
                    preferred_element_type=jnp.float32) + cl_b_ref[...]
    log_ref[0] = jnp.where(mf > 0.0, logit, -jnp.inf)


def _relation(obj, g1, ub, maskf, tabg, dk, sp_w3, sp_b2, pr_w3, pr_b2,
              sc_w, vl_w, vl_b2, fu_w1a, langc, fu_b12, fu_w2, fu_b22,
              cl_w, cl_b2):
    grid = (B, N // TN)
    return pl.pallas_call(
        _rel_body,
        grid=grid,
        in_specs=[
            pl.BlockSpec((1, TN, D_OBJ), lambda b, i: (b, i, 0)),
            pl.BlockSpec((1, TN, D_RH), lambda b, i: (b, i, 0)),
            pl.BlockSpec((1, TN, D_RH), lambda b, i: (b, i, 0)),
            pl.BlockSpec((1, TN, 1), lambda b, i: (b, i, 0)),
            pl.BlockSpec((1, TN * K, D_RH), lambda b, i: (b, i, 0)),
            pl.BlockSpec((1, TN * K, 1), lambda b, i: (b, i, 0)),
            pl.BlockSpec((1, D_RH), lambda b, i: (0, 0)),
            pl.BlockSpec((1, D_RH), lambda b, i: (0, 0)),
            pl.BlockSpec((D_RH, D_RH), lambda b, i: (0, 0)),
            pl.BlockSpec((1, D_RH), lambda b, i: (0, 0)),
            pl.BlockSpec((D_RH + D_LANG, 1), lambda b, i: (0, 0)),
            pl.BlockSpec((D_RH, D_OBJ), lambda b, i: (0, 0)),
            pl.BlockSpec((1, D_OBJ), lambda b, i: (0, 0)),
            pl.BlockSpec((D_OBJ, D_FUS), lambda b, i: (0, 0)),
            pl.BlockSpec((1, 1, D_FUS), lambda b, i: (b, 0, 0)),
            pl.BlockSpec((1, D_FUS), lambda b, i: (0, 0)),
            pl.BlockSpec((D_FUS, D_FUS), lambda b, i: (0, 0)),
            pl.BlockSpec((1, D_FUS), lambda b, i: (0, 0)),
            pl.BlockSpec((D_FUS, 1), lambda b, i: (0, 0)),
            pl.BlockSpec((1, 1), lambda b, i: (0, 0)),
        ],
        out_specs=[
            pl.BlockSpec((1, TN * K, 1), lambda b, i: (b, i, 0)),
            pl.BlockSpec((1, TN, D_OBJ), lambda b, i: (b, i, 0)),
            pl.BlockSpec((1, TN, 1), lambda b, i: (b, i, 0)),
        ],
        out_shape=[
            jax.ShapeDtypeStruct((B, N * K, 1), jnp.float32),
            jax.ShapeDtypeStruct((B, N, D_OBJ), jnp.float32),
            jax.ShapeDtypeStruct((B, N, 1), jnp.float32),
        ],
    )(obj, g1, ub, maskf, tabg, dk, sp_w3, sp_b2, pr_w3, pr_b2, sc_w,
      vl_w, vl_b2, fu_w1a, langc, fu_b12, fu_w2, fu_b22, cl_w, cl_b2)


def kernel(points, object_mask, text_features, centers, sizes, pe_w1, pe_b1,
           pe_w2, pe_b2, le_w1, le_b1, le_w2, le_b2, sp_w, sp_b, pr_w, pr_b,
           sc_w, sc_b, vl_w, vl_b, fu_w1, fu_b1, fu_w2, fu_b2, cl_w, cl_b):
    maskf = object_mask.astype(jnp.float32)[:, :, None]
    maskf_row = object_mask.astype(jnp.float32)[:, None, :]
    centers_t = jnp.transpose(centers, (0, 2, 1))

    obj, g1, tab, ub, nbr, nbrg, dval, lang, langc = _encode(
        points, centers, sizes, maskf, centers_t, maskf_row,
        pe_w1, pe_b1[None, :], pe_w2, pe_b2[None, :],
        pr_w[0:D_OBJ], pr_w[D_OBJ:2 * D_OBJ], sp_w,
        text_features, le_w1, le_b1[None, :], le_w2, le_b2[None, :],
        fu_w1[D_OBJ:])

    tabg = _sc_gather(nbrg.reshape(-1), tab.reshape(B * N, D_RH))

    wgt2, enhanced, logits3 = _relation(
        obj, g1, ub, maskf, tabg.reshape(B, N * K, D_RH),
        dval.reshape(B, N * K, 1), sp_w[3:4], sp_b[None, :],
        pr_w[2 * D_OBJ:], pr_b[None, :], sc_w, vl_w, vl_b[None, :],
        fu_w1[0:D_OBJ], langc[:, None, :], fu_b1[None, :], fu_w2,
        fu_b2[None, :],
        cl_w, cl_b[None, :])

    wgt = wgt2.reshape(B, N, K)
    logits = logits3[:, :, 0]
    return (logits, enhanced, obj, lang, wgt, nbr)


# speedup vs baseline: 1.4207x; 1.0309x over previous
import functools

import jax
import jax.numpy as jnp
from jax import lax
from jax.experimental import pallas as pl
from jax.experimental.pallas import tpu as pltpu
from jax.experimental.pallas import tpu_sc as plsc

B, N, K = 8, 1024, 8
D_PT, D_PH, D_OBJ = 256, 128, 256
D_LIN, D_LH, D_LANG = 768, 256, 256
D_RH = 128
D_FUS = 512
DIAM = 5.0

TA = 1024
TN = 512

SC_NC, SC_NS = 2, 16
SC_NW = SC_NC * SC_NS
SC_ROWS = B * N * K
SC_RPW = SC_ROWS // SC_NW
SC_CH = 128
SC_NCH = SC_RPW // SC_CH


def _enc_body(pts_ref, cen_ref, siz_ref, mf_ref, cent_ref, mrow_ref,
              pe_w1_ref, pe_b1_ref, pe_w2_ref, pe_b2_ref,
              pr_w1_ref, pr_w2_ref, sp_w_ref,
              txt_ref, le_w1_ref, le_b1_ref, le_w2_ref, le_b2_ref,
              fu_w1b_ref,
              obj_ref, g1_ref, tab_ref, ub_ref,
              nbr_ref, nbrg_ref, dval_ref, lang_ref, langc_ref):
    @pl.when(jnp.logical_and(pl.program_id(0) == 0, pl.program_id(1) == 0))
    def _():
        hl = jnp.maximum(jnp.dot(txt_ref[...], le_w1_ref[...],
                                 preferred_element_type=jnp.float32)
                         + le_b1_ref[...], 0.0)
        lang = jnp.dot(hl, le_w2_ref[...],
                       preferred_element_type=jnp.float32) + le_b2_ref[...]
        lang_ref[...] = lang
        langc_ref[...] = jnp.dot(lang, fu_w1b_ref[...],
                                 preferred_element_type=jnp.float32)
    b = pl.program_id(0)
    i = pl.program_id(1)
    cb = cen_ref[0]
    ct = cent_ref[0]
    dx = cb[:, 0:1] - ct[0:1, :]
    dy = cb[:, 1:2] - ct[1:2, :]
    dz = cb[:, 2:3] - ct[2:3, :]
    sq = dx * dx + dy * dy + dz * dz
    col = lax.broadcasted_iota(jnp.int32, (TA, N), 1)
    row = i * TA + lax.broadcasted_iota(jnp.int32, (TA, N), 0)
    valid = (mrow_ref[0] > 0.0) & (col != row)
    dm = jnp.where(valid, sq, jnp.inf)
    idxs = []
    minvs = []
    for k in range(K):
        minv = jnp.min(dm, axis=1, keepdims=True)
        idx = jnp.min(jnp.where(dm == minv, col, N), axis=1, keepdims=True)
        idxs.append(idx)
        minvs.append(minv)
        dm = jnp.where(col == idx, jnp.inf, dm)
    nbr = jnp.concatenate(idxs, axis=1)
    nbr_ref[0] = nbr
    nbrg_ref[0] = nbr + b * N
    dval_ref[0] = jnp.sqrt(jnp.concatenate(minvs, axis=1) + 1e-12) / DIAM

    pts = pts_ref[0]
    h = jnp.maximum(jnp.dot(pts, pe_w1_ref[...],
                            preferred_element_type=jnp.float32)
                    + pe_b1_ref[...], 0.0)
    obj = (jnp.dot(h, pe_w2_ref[...], preferred_element_type=jnp.float32)
           + pe_b2_ref[...]) * mf_ref[0]
    obj_ref[0] = obj
    g1_ref[0] = jnp.dot(obj, pr_w1_ref[...], preferred_element_type=jnp.float32)
    cw = jnp.dot(cen_ref[0], sp_w_ref[0:3, :],
                 preferred_element_type=jnp.float32) * (1.0 / DIAM)
    g2 = jnp.dot(obj, pr_w2_ref[...], preferred_element_type=jnp.float32)
    ua = cw + jnp.dot(siz_ref[0], sp_w_ref[7:10, :],
                      preferred_element_type=jnp.float32)
    g2i = lax.bitcast_convert_type(g2, jnp.int32)
    uai = lax.bitcast_convert_type(ua, jnp.int32)
    hi = (g2i + 0x8000) & jnp.int32(-65536)
    lo = jnp.right_shift(uai + 0x8000, 16) & 0xFFFF
    tab_ref[0] = hi | lo
    ub_ref[0] = jnp.dot(siz_ref[0], sp_w_ref[4:7, :],
                        preferred_element_type=jnp.float32) - cw


def _encode(points, centers, sizes, maskf, centers_t, maskf_row,
            pe_w1, pe_b1, pe_w2, pe_b2, pr_w1, pr_w2, sp_w,
            text_features, le_w1, le_b1, le_w2, le_b2, fu_w1b):
    grid = (B, N // TA)
    return pl.pallas_call(
        _enc_body,
        grid=grid,
        in_specs=[
            pl.BlockSpec((1, TA, D_PT), lambda b, i: (b, i, 0)),
            pl.BlockSpec((1, TA, 3), lambda b, i: (b, i, 0)),
            pl.BlockSpec((1, TA, 3), lambda b, i: (b, i, 0)),
            pl.BlockSpec((1, TA, 1), lambda b, i: (b, i, 0)),
            pl.BlockSpec((1, 3, N), lambda b, i: (b, 0, 0)),
            pl.BlockSpec((1, 1, N), lambda b, i: (b, 0, 0)),
            pl.BlockSpec((D_PT, D_PH), lambda b, i: (0, 0)),
            pl.BlockSpec((1, D_PH), lambda b, i: (0, 0)),
            pl.BlockSpec((D_PH, D_OBJ), lambda b, i: (0, 0)),
            pl.BlockSpec((1, D_OBJ), lambda b, i: (0, 0)),
            pl.BlockSpec((D_OBJ, D_RH), lambda b, i: (0, 0)),
            pl.BlockSpec((D_OBJ, D_RH), lambda b, i: (0, 0)),
            pl.BlockSpec((10, D_RH), lambda b, i: (0, 0)),
            pl.BlockSpec((B, D_LIN), lambda b, i: (0, 0)),
            pl.BlockSpec((D_LIN, D_LH), lambda b, i: (0, 0)),
            pl.BlockSpec((1, D_LH), lambda b, i: (0, 0)),
            pl.BlockSpec((D_LH, D_LANG), lambda b, i: (0, 0)),
            pl.BlockSpec((1, D_LANG), lambda b, i: (0, 0)),
            pl.BlockSpec((D_LANG, D_FUS), lambda b, i: (0, 0)),
        ],
        out_specs=[
            pl.BlockSpec((1, TA, D_OBJ), lambda b, i: (b, i, 0)),
            pl.BlockSpec((1, TA, D_RH), lambda b, i: (b, i, 0)),
            pl.BlockSpec((1, TA, D_RH), lambda b, i: (b, i, 0)),
            pl.BlockSpec((1, TA, D_RH), lambda b, i: (b, i, 0)),
            pl.BlockSpec((1, TA, K), lambda b, i: (b, i, 0)),
            pl.BlockSpec((1, TA, K), lambda b, i: (b, i, 0)),
            pl.BlockSpec((1, TA, K), lambda b, i: (b, i, 0)),
            pl.BlockSpec((B, D_LANG), lambda b, i: (0, 0)),
            pl.BlockSpec((B, D_FUS), lambda b, i: (0, 0)),
        ],
        out_shape=[
            jax.ShapeDtypeStruct((B, N, D_OBJ), jnp.float32),
            jax.ShapeDtypeStruct((B, N, D_RH), jnp.float32),
            jax.ShapeDtypeStruct((B, N, D_RH), jnp.int32),
            jax.ShapeDtypeStruct((B, N, D_RH), jnp.float32),
            jax.ShapeDtypeStruct((B, N, K), jnp.int32),
            jax.ShapeDtypeStruct((B, N, K), jnp.int32),
            jax.ShapeDtypeStruct((B, N, K), jnp.float32),
            jax.ShapeDtypeStruct((B, D_LANG), jnp.float32),
            jax.ShapeDtypeStruct((B, D_FUS), jnp.float32),
        ],
    )(points, centers, sizes, maskf, centers_t, maskf_row,
      pe_w1, pe_b1, pe_w2, pe_b2, pr_w1, pr_w2, sp_w,
      text_features, le_w1, le_b1, le_w2, le_b2, fu_w1b)


def _sc_gather_body(idx_hbm, tab_hbm, out_hbm, idx_v,
                    buf0, buf1, gs0, gs1, ws0, ws1):
    wid = lax.axis_index("s") * SC_NC + lax.axis_index("c")
    base = wid * SC_RPW
    pltpu.sync_copy(idx_hbm.at[pl.ds(base, SC_RPW)], idx_v)
    bufs = (buf0, buf1)
    gsem = (gs0, gs1)
    wsem = (ws0, ws1)
    gops = [None, None]
    wops = [None, None]
    for ci in range(SC_NCH):
        s = ci % 2
        if wops[s] is not None:
            wops[s].wait()
        gops[s] = pltpu.async_copy(
            tab_hbm.at[idx_v.at[pl.ds(ci * SC_CH, SC_CH)]], bufs[s], gsem[s])
        p = 1 - s
        if gops[p] is not None:
            gops[p].wait()
            wops[p] = pltpu.async_copy(
                bufs[p], out_hbm.at[pl.ds(base + (ci - 1) * SC_CH, SC_CH)],
                wsem[p])
    last = (SC_NCH - 1) % 2
    gops[last].wait()
    wops[last] = pltpu.async_copy(
        bufs[last], out_hbm.at[pl.ds(base + (SC_NCH - 1) * SC_CH, SC_CH)],
        wsem[last])
    wops[1 - last].wait()
    wops[last].wait()


@functools.cache
def _sc_gather_kernel():
    return pl.kernel(
        _sc_gather_body,
        out_type=jax.ShapeDtypeStruct((SC_ROWS, D_RH), jnp.int32),
        mesh=plsc.VectorSubcoreMesh(core_axis_name="c", subcore_axis_name="s",
                                    num_cores=SC_NC, num_subcores=SC_NS),
        scratch_types=[
            pltpu.VMEM((SC_RPW,), jnp.int32),
            pltpu.VMEM((SC_CH, D_RH), jnp.int32),
            pltpu.VMEM((SC_CH, D_RH), jnp.int32),
            pltpu.SemaphoreType.DMA,
            pltpu.SemaphoreType.DMA,
            pltpu.SemaphoreType.DMA,
            pltpu.SemaphoreType.DMA,
        ],
    )


def _sc_gather(nbrg_flat, tab):
    return _sc_gather_kernel()(nbrg_flat, tab)


def _rel_body(obj_ref, g1_ref, ub_ref, mf_ref, tabg_ref, dk_ref,
              sp_w3_ref, sp_b_ref, pr_w3_ref, pr_b_ref, sc_w_ref,
              vl_w_ref, vl_b_ref, fu_w1a_ref, langc_ref, fu_b1_ref,
              fu_w2_ref, fu_b2_ref, cl_w_ref, cl_b_ref,
              wgt_ref, enh_ref, log_ref):
    tabg3 = tabg_ref[0].reshape(TN, K, D_RH)
    g2j3 = lax.bitcast_convert_type(tabg3 & jnp.int32(-65536), jnp.float32)
    uaj3 = lax.bitcast_convert_type(jnp.left_shift(tabg3, 16), jnp.float32)
    hdk = jnp.dot(dk_ref[0], sp_w3_ref[...],
                  preferred_element_type=jnp.float32)
    hsp3 = jnp.maximum(
        uaj3
        + ub_ref[0].reshape(TN, 1, D_RH)
        + hdk.reshape(TN, K, D_RH)
        + sp_b_ref[...].reshape(1, 1, D_RH), 0.0)

    hp = jnp.dot(hsp3.reshape(TN * K, D_RH), pr_w3_ref[...],
                 preferred_element_type=jnp.float32)
    pair3 = jnp.maximum(
        g1_ref[0].reshape(TN, 1, D_RH)
        + g2j3
        + hp.reshape(TN, K, D_RH)
        + pr_b_ref[...].reshape(1, 1, D_RH), 0.0)
    pair2 = pair3.reshape(TN * K, D_RH)

    score_c = jnp.dot(pair2, sc_w_ref[0:D_RH, :],
                      preferred_element_type=jnp.float32).reshape(TN, K, 1)
    mc = jnp.max(score_c, axis=1, keepdims=True)
    ec = jnp.exp(score_c - mc)
    sc = jnp.sum(ec, axis=1, keepdims=True)
    wgtc = ec / sc
    wgt_ref[0] = wgtc.reshape(TN * K, 1)

    wgt_rep = jnp.dot(wgtc.reshape(TN * K, 1), jnp.ones((1, D_RH), jnp.float32),
                      preferred_element_type=jnp.float32)
    pbar = jnp.sum(wgt_rep.reshape(TN, K, D_RH) * pair3, axis=1)
    ctx = jnp.dot(pbar, vl_w_ref[...],
                  preferred_element_type=jnp.float32) + vl_b_ref[...]
    mf = mf_ref[0]
    enh = (obj_ref[0] + ctx) * mf
    enh_ref[0] = enh

    f1 = jnp.maximum(jnp.dot(enh, fu_w1a_ref[...],
                             preferred_element_type=jnp.float32)
                     + langc_ref[0] + fu_b1_ref[...], 0.0)
    f2 = jnp.maximum(jnp.dot(f1, fu_w2_ref[...],
                             preferred_element_type=jnp.float32)
                     + fu_b2_ref[...], 0.0)
    logit = jnp.dot(f2, cl_w_ref[...],
                    preferred_element_type=jnp.float32) + cl_b_ref[...]
    log_ref[0] = jnp.where(mf > 0.0, logit, -jnp.inf)


def _relation(obj, g1, ub, maskf, tabg, dk, sp_w3, sp_b2, pr_w3, pr_b2,
              sc_w, vl_w, vl_b2, fu_w1a, langc, fu_b12, fu_w2, fu_b22,
              cl_w, cl_b2):
    grid = (B, N // TN)
    return pl.pallas_call(
        _rel_body,
        grid=grid,
        in_specs=[
            pl.BlockSpec((1, TN, D_OBJ), lambda b, i: (b, i, 0)),
            pl.BlockSpec((1, TN, D_RH), lambda b, i: (b, i, 0)),
            pl.BlockSpec((1, TN, D_RH), lambda b, i: (b, i, 0)),
            pl.BlockSpec((1, TN, 1), lambda b, i: (b, i, 0)),
            pl.BlockSpec((1, TN * K, D_RH), lambda b, i: (b, i, 0)),
            pl.BlockSpec((1, TN * K, 1), lambda b, i: (b, i, 0)),
            pl.BlockSpec((1, D_RH), lambda b, i: (0, 0)),
            pl.BlockSpec((1, D_RH), lambda b, i: (0, 0)),
            pl.BlockSpec((D_RH, D_RH), lambda b, i: (0, 0)),
            pl.BlockSpec((1, D_RH), lambda b, i: (0, 0)),
            pl.BlockSpec((D_RH + D_LANG, 1), lambda b, i: (0, 0)),
            pl.BlockSpec((D_RH, D_OBJ), lambda b, i: (0, 0)),
            pl.BlockSpec((1, D_OBJ), lambda b, i: (0, 0)),
            pl.BlockSpec((D_OBJ, D_FUS), lambda b, i: (0, 0)),
            pl.BlockSpec((1, 1, D_FUS), lambda b, i: (b, 0, 0)),
            pl.BlockSpec((1, D_FUS), lambda b, i: (0, 0)),
            pl.BlockSpec((D_FUS, D_FUS), lambda b, i: (0, 0)),
            pl.BlockSpec((1, D_FUS), lambda b, i: (0, 0)),
            pl.BlockSpec((D_FUS, 1), lambda b, i: (0, 0)),
            pl.BlockSpec((1, 1), lambda b, i: (0, 0)),
        ],
        out_specs=[
            pl.BlockSpec((1, TN * K, 1), lambda b, i: (b, i, 0)),
            pl.BlockSpec((1, TN, D_OBJ), lambda b, i: (b, i, 0)),
            pl.BlockSpec((1, TN, 1), lambda b, i: (b, i, 0)),
        ],
        out_shape=[
            jax.ShapeDtypeStruct((B, N * K, 1), jnp.float32),
            jax.ShapeDtypeStruct((B, N, D_OBJ), jnp.float32),
            jax.ShapeDtypeStruct((B, N, 1), jnp.float32),
        ],
    )(obj, g1, ub, maskf, tabg, dk, sp_w3, sp_b2, pr_w3, pr_b2, sc_w,
      vl_w, vl_b2, fu_w1a, langc, fu_b12, fu_w2, fu_b22, cl_w, cl_b2)


def kernel(points, object_mask, text_features, centers, sizes, pe_w1, pe_b1,
           pe_w2, pe_b2, le_w1, le_b1, le_w2, le_b2, sp_w, sp_b, pr_w, pr_b,
           sc_w, sc_b, vl_w, vl_b, fu_w1, fu_b1, fu_w2, fu_b2, cl_w, cl_b):
    maskf = object_mask.astype(jnp.float32)[:, :, None]
    maskf_row = object_mask.astype(jnp.float32)[:, None, :]
    centers_t = jnp.transpose(centers, (0, 2, 1))

    obj, g1, tab, ub, nbr, nbrg, dval, lang, langc = _encode(
        points, centers, sizes, maskf, centers_t, maskf_row,
        pe_w1, pe_b1[None, :], pe_w2, pe_b2[None, :],
        pr_w[0:D_OBJ], pr_w[D_OBJ:2 * D_OBJ], sp_w,
        text_features, le_w1, le_b1[None, :], le_w2, le_b2[None, :],
        fu_w1[D_OBJ:])

    tabg = _sc_gather(nbrg.reshape(-1), tab.reshape(B * N, D_RH))

    wgt2, enhanced, logits3 = _relation(
        obj, g1, ub, maskf, tabg.reshape(B, N * K, D_RH),
        dval.reshape(B, N * K, 1), sp_w[3:4], sp_b[None, :],
        pr_w[2 * D_OBJ:], pr_b[None, :], sc_w, vl_w, vl_b[None, :],
        fu_w1[0:D_OBJ], langc[:, None, :], fu_b1[None, :], fu_w2,
        fu_b2[None, :],
        cl_w, cl_b[None, :])

    wgt = wgt2.reshape(B, N, K)
    logits = logits3[:, :, 0]
    return (logits, enhanced, obj, lang, wgt, nbr)


# speedup vs baseline: 1.4328x; 1.0085x over previous
import functools

import jax
import jax.numpy as jnp
from jax import lax
from jax.experimental import pallas as pl
from jax.experimental.pallas import tpu as pltpu
from jax.experimental.pallas import tpu_sc as plsc

B, N, K = 8, 1024, 8
D_PT, D_PH, D_OBJ = 256, 128, 256
D_LIN, D_LH, D_LANG = 768, 256, 256
D_RH = 128
D_FUS = 512
DIAM = 5.0

TA = 1024
TN = 1024

SC_NC, SC_NS = 2, 16
SC_NW = SC_NC * SC_NS
SC_ROWS = B * N * K
SC_RPW = SC_ROWS // SC_NW
SC_CH = 128
SC_NCH = SC_RPW // SC_CH


def _enc_body(pts_ref, cen_ref, siz_ref, mf_ref, cent_ref, mrow_ref,
              pe_w1_ref, pe_b1_ref, pe_w2_ref, pe_b2_ref,
              pr_w1_ref, pr_w2_ref, sp_w_ref,
              txt_ref, le_w1_ref, le_b1_ref, le_w2_ref, le_b2_ref,
              fu_w1b_ref,
              obj_ref, g1_ref, tab_ref, ub_ref,
              nbr_ref, nbrg_ref, dval_ref, lang_ref, langc_ref):
    @pl.when(jnp.logical_and(pl.program_id(0) == 0, pl.program_id(1) == 0))
    def _():
        hl = jnp.maximum(jnp.dot(txt_ref[...], le_w1_ref[...],
                                 preferred_element_type=jnp.float32)
                         + le_b1_ref[...], 0.0)
        lang = jnp.dot(hl, le_w2_ref[...],
                       preferred_element_type=jnp.float32) + le_b2_ref[...]
        lang_ref[...] = lang
        langc_ref[...] = jnp.dot(lang, fu_w1b_ref[...],
                                 preferred_element_type=jnp.float32)
    b = pl.program_id(0)
    i = pl.program_id(1)
    cb = cen_ref[0]
    ct = cent_ref[0]
    dx = cb[:, 0:1] - ct[0:1, :]
    dy = cb[:, 1:2] - ct[1:2, :]
    dz = cb[:, 2:3] - ct[2:3, :]
    sq = dx * dx + dy * dy + dz * dz
    col = lax.broadcasted_iota(jnp.int32, (TA, N), 1)
    row = i * TA + lax.broadcasted_iota(jnp.int32, (TA, N), 0)
    valid = (mrow_ref[0] > 0.0) & (col != row)
    dm = jnp.where(valid, sq, jnp.inf)
    idxs = []
    minvs = []
    for k in range(K):
        minv = jnp.min(dm, axis=1, keepdims=True)
        idx = jnp.min(jnp.where(dm == minv, col, N), axis=1, keepdims=True)
        idxs.append(idx)
        minvs.append(minv)
        dm = jnp.where(col == idx, jnp.inf, dm)
    nbr = jnp.concatenate(idxs, axis=1)
    nbr_ref[0] = nbr
    nbrg_ref[0] = nbr + b * N
    dval_ref[0] = jnp.sqrt(jnp.concatenate(minvs, axis=1) + 1e-12) / DIAM

    pts = pts_ref[0]
    h = jnp.maximum(jnp.dot(pts, pe_w1_ref[...],
                            preferred_element_type=jnp.float32)
                    + pe_b1_ref[...], 0.0)
    obj = (jnp.dot(h, pe_w2_ref[...], preferred_element_type=jnp.float32)
           + pe_b2_ref[...]) * mf_ref[0]
    obj_ref[0] = obj
    g1_ref[0] = jnp.dot(obj, pr_w1_ref[...], preferred_element_type=jnp.float32)
    cw = jnp.dot(cen_ref[0], sp_w_ref[0:3, :],
                 preferred_element_type=jnp.float32) * (1.0 / DIAM)
    g2 = jnp.dot(obj, pr_w2_ref[...], preferred_element_type=jnp.float32)
    ua = cw + jnp.dot(siz_ref[0], sp_w_ref[7:10, :],
                      preferred_element_type=jnp.float32)
    g2i = lax.bitcast_convert_type(g2, jnp.int32)
    uai = lax.bitcast_convert_type(ua, jnp.int32)
    hi = (g2i + 0x8000) & jnp.int32(-65536)
    lo = jnp.right_shift(uai + 0x8000, 16) & 0xFFFF
    tab_ref[0] = hi | lo
    ub_ref[0] = jnp.dot(siz_ref[0], sp_w_ref[4:7, :],
                        preferred_element_type=jnp.float32) - cw


def _encode(points, centers, sizes, maskf, centers_t, maskf_row,
            pe_w1, pe_b1, pe_w2, pe_b2, pr_w1, pr_w2, sp_w,
            text_features, le_w1, le_b1, le_w2, le_b2, fu_w1b):
    grid = (B, N // TA)
    return pl.pallas_call(
        _enc_body,
        grid=grid,
        in_specs=[
            pl.BlockSpec((1, TA, D_PT), lambda b, i: (b, i, 0)),
            pl.BlockSpec((1, TA, 3), lambda b, i: (b, i, 0)),
            pl.BlockSpec((1, TA, 3), lambda b, i: (b, i, 0)),
            pl.BlockSpec((1, TA, 1), lambda b, i: (b, i, 0)),
            pl.BlockSpec((1, 3, N), lambda b, i: (b, 0, 0)),
            pl.BlockSpec((1, 1, N), lambda b, i: (b, 0, 0)),
            pl.BlockSpec((D_PT, D_PH), lambda b, i: (0, 0)),
            pl.BlockSpec((1, D_PH), lambda b, i: (0, 0)),
            pl.BlockSpec((D_PH, D_OBJ), lambda b, i: (0, 0)),
            pl.BlockSpec((1, D_OBJ), lambda b, i: (0, 0)),
            pl.BlockSpec((D_OBJ, D_RH), lambda b, i: (0, 0)),
            pl.BlockSpec((D_OBJ, D_RH), lambda b, i: (0, 0)),
            pl.BlockSpec((10, D_RH), lambda b, i: (0, 0)),
            pl.BlockSpec((B, D_LIN), lambda b, i: (0, 0)),
            pl.BlockSpec((D_LIN, D_LH), lambda b, i: (0, 0)),
            pl.BlockSpec((1, D_LH), lambda b, i: (0, 0)),
            pl.BlockSpec((D_LH, D_LANG), lambda b, i: (0, 0)),
            pl.BlockSpec((1, D_LANG), lambda b, i: (0, 0)),
            pl.BlockSpec((D_LANG, D_FUS), lambda b, i: (0, 0)),
        ],
        out_specs=[
            pl.BlockSpec((1, TA, D_OBJ), lambda b, i: (b, i, 0)),
            pl.BlockSpec((1, TA, D_RH), lambda b, i: (b, i, 0)),
            pl.BlockSpec((1, TA, D_RH), lambda b, i: (b, i, 0)),
            pl.BlockSpec((1, TA, D_RH), lambda b, i: (b, i, 0)),
            pl.BlockSpec((1, TA, K), lambda b, i: (b, i, 0)),
            pl.BlockSpec((1, TA, K), lambda b, i: (b, i, 0)),
            pl.BlockSpec((1, TA, K), lambda b, i: (b, i, 0)),
            pl.BlockSpec((B, D_LANG), lambda b, i: (0, 0)),
            pl.BlockSpec((B, D_FUS), lambda b, i: (0, 0)),
        ],
        out_shape=[
            jax.ShapeDtypeStruct((B, N, D_OBJ), jnp.float32),
            jax.ShapeDtypeStruct((B, N, D_RH), jnp.float32),
            jax.ShapeDtypeStruct((B, N, D_RH), jnp.int32),
            jax.ShapeDtypeStruct((B, N, D_RH), jnp.float32),
            jax.ShapeDtypeStruct((B, N, K), jnp.int32),
            jax.ShapeDtypeStruct((B, N, K), jnp.int32),
            jax.ShapeDtypeStruct((B, N, K), jnp.float32),
            jax.ShapeDtypeStruct((B, D_LANG), jnp.float32),
            jax.ShapeDtypeStruct((B, D_FUS), jnp.float32),
        ],
    )(points, centers, sizes, maskf, centers_t, maskf_row,
      pe_w1, pe_b1, pe_w2, pe_b2, pr_w1, pr_w2, sp_w,
      text_features, le_w1, le_b1, le_w2, le_b2, fu_w1b)


def _sc_gather_body(idx_hbm, tab_hbm, out_hbm, idx_v,
                    buf0, buf1, gs0, gs1, ws0, ws1):
    wid = lax.axis_index("s") * SC_NC + lax.axis_index("c")
    base = wid * SC_RPW
    pltpu.sync_copy(idx_hbm.at[pl.ds(base, SC_RPW)], idx_v)
    bufs = (buf0, buf1)
    gsem = (gs0, gs1)
    wsem = (ws0, ws1)
    gops = [None, None]
    wops = [None, None]
    for ci in range(SC_NCH):
        s = ci % 2
        if wops[s] is not None:
            wops[s].wait()
        gops[s] = pltpu.async_copy(
            tab_hbm.at[idx_v.at[pl.ds(ci * SC_CH, SC_CH)]], bufs[s], gsem[s])
        p = 1 - s
        if gops[p] is not None:
            gops[p].wait()
            wops[p] = pltpu.async_copy(
                bufs[p], out_hbm.at[pl.ds(base + (ci - 1) * SC_CH, SC_CH)],
                wsem[p])
    last = (SC_NCH - 1) % 2
    gops[last].wait()
    wops[last] = pltpu.async_copy(
        bufs[last], out_hbm.at[pl.ds(base + (SC_NCH - 1) * SC_CH, SC_CH)],
        wsem[last])
    wops[1 - last].wait()
    wops[last].wait()


@functools.cache
def _sc_gather_kernel():
    return pl.kernel(
        _sc_gather_body,
        out_type=jax.ShapeDtypeStruct((SC_ROWS, D_RH), jnp.int32),
        mesh=plsc.VectorSubcoreMesh(core_axis_name="c", subcore_axis_name="s",
                                    num_cores=SC_NC, num_subcores=SC_NS),
        scratch_types=[
            pltpu.VMEM((SC_RPW,), jnp.int32),
            pltpu.VMEM((SC_CH, D_RH), jnp.int32),
            pltpu.VMEM((SC_CH, D_RH), jnp.int32),
            pltpu.SemaphoreType.DMA,
            pltpu.SemaphoreType.DMA,
            pltpu.SemaphoreType.DMA,
            pltpu.SemaphoreType.DMA,
        ],
    )


def _sc_gather(nbrg_flat, tab):
    return _sc_gather_kernel()(nbrg_flat, tab)


def _rel_body(obj_ref, g1_ref, ub_ref, mf_ref, tabg_ref, dk_ref,
              sp_w3_ref, sp_b_ref, pr_w3_ref, pr_b_ref, sc_w_ref,
              vl_w_ref, vl_b_ref, fu_w1a_ref, langc_ref, fu_b1_ref,
              fu_w2_ref, fu_b2_ref, cl_w_ref, cl_b_ref,
              wgt_ref, enh_ref, log_ref):
    tabg3 = tabg_ref[0].reshape(TN, K, D_RH)
    g2j3 = lax.bitcast_convert_type(tabg3 & jnp.int32(-65536), jnp.float32)
    uaj3 = lax.bitcast_convert_type(jnp.left_shift(tabg3, 16), jnp.float32)
    hdk = jnp.dot(dk_ref[0], sp_w3_ref[...],
                  preferred_element_type=jnp.float32)
    hsp3 = jnp.maximum(
        uaj3
        + ub_ref[0].reshape(TN, 1, D_RH)
        + hdk.reshape(TN, K, D_RH)
        + sp_b_ref[...].reshape(1, 1, D_RH), 0.0)

    hp = jnp.dot(hsp3.reshape(TN * K, D_RH), pr_w3_ref[...],
                 preferred_element_type=jnp.float32)
    pair3 = jnp.maximum(
        g1_ref[0].reshape(TN, 1, D_RH)
        + g2j3
        + hp.reshape(TN, K, D_RH)
        + pr_b_ref[...].reshape(1, 1, D_RH), 0.0)
    pair2 = pair3.reshape(TN * K, D_RH)

    score_c = jnp.dot(pair2, sc_w_ref[0:D_RH, :],
                      preferred_element_type=jnp.float32).reshape(TN, K, 1)
    mc = jnp.max(score_c, axis=1, keepdims=True)
    ec = jnp.exp(score_c - mc)
    sc = jnp.sum(ec, axis=1, keepdims=True)
    wgtc = ec / sc
    wgt_ref[0] = wgtc.reshape(TN * K, 1)

    wgt_rep = jnp.dot(wgtc.reshape(TN * K, 1), jnp.ones((1, D_RH), jnp.float32),
                      preferred_element_type=jnp.float32)
    pbar = jnp.sum(wgt_rep.reshape(TN, K, D_RH) * pair3, axis=1)
    ctx = jnp.dot(pbar, vl_w_ref[...],
                  preferred_element_type=jnp.float32) + vl_b_ref[...]
    mf = mf_ref[0]
    enh = (obj_ref[0] + ctx) * mf
    enh_ref[0] = enh

    f1 = jnp.maximum(jnp.dot(enh, fu_w1a_ref[...],
                             preferred_element_type=jnp.float32)
                     + langc_ref[0] + fu_b1_ref[...], 0.0)
    f2 = jnp.maximum(jnp.dot(f1, fu_w2_ref[...],
                             preferred_element_type=jnp.float32)
                     + fu_b2_ref[...], 0.0)
    logit = jnp.dot(f2, cl_w_ref[...],
                    preferred_element_type=jnp.float32) + cl_b_ref[...]
    log_ref[0] = jnp.where(mf > 0.0, logit, -jnp.inf)


def _relation(obj, g1, ub, maskf, tabg, dk, sp_w3, sp_b2, pr_w3, pr_b2,
              sc_w, vl_w, vl_b2, fu_w1a, langc, fu_b12, fu_w2, fu_b22,
              cl_w, cl_b2):
    grid = (B, N // TN)
    return pl.pallas_call(
        _rel_body,
        grid=grid,
        in_specs=[
            pl.BlockSpec((1, TN, D_OBJ), lambda b, i: (b, i, 0)),
            pl.BlockSpec((1, TN, D_RH), lambda b, i: (b, i, 0)),
            pl.BlockSpec((1, TN, D_RH), lambda b, i: (b, i, 0)),
            pl.BlockSpec((1, TN, 1), lambda b, i: (b, i, 0)),
            pl.BlockSpec((1, TN * K, D_RH), lambda b, i: (b, i, 0)),
            pl.BlockSpec((1, TN * K, 1), lambda b, i: (b, i, 0)),
            pl.BlockSpec((1, D_RH), lambda b, i: (0, 0)),
            pl.BlockSpec((1, D_RH), lambda b, i: (0, 0)),
            pl.BlockSpec((D_RH, D_RH), lambda b, i: (0, 0)),
            pl.BlockSpec((1, D_RH), lambda b, i: (0, 0)),
            pl.BlockSpec((D_RH + D_LANG, 1), lambda b, i: (0, 0)),
            pl.BlockSpec((D_RH, D_OBJ), lambda b, i: (0, 0)),
            pl.BlockSpec((1, D_OBJ), lambda b, i: (0, 0)),
            pl.BlockSpec((D_OBJ, D_FUS), lambda b, i: (0, 0)),
            pl.BlockSpec((1, 1, D_FUS), lambda b, i: (b, 0, 0)),
            pl.BlockSpec((1, D_FUS), lambda b, i: (0, 0)),
            pl.BlockSpec((D_FUS, D_FUS), lambda b, i: (0, 0)),
            pl.BlockSpec((1, D_FUS), lambda b, i: (0, 0)),
            pl.BlockSpec((D_FUS, 1), lambda b, i: (0, 0)),
            pl.BlockSpec((1, 1), lambda b, i: (0, 0)),
        ],
        out_specs=[
            pl.BlockSpec((1, TN * K, 1), lambda b, i: (b, i, 0)),
            pl.BlockSpec((1, TN, D_OBJ), lambda b, i: (b, i, 0)),
            pl.BlockSpec((1, TN, 1), lambda b, i: (b, i, 0)),
        ],
        out_shape=[
            jax.ShapeDtypeStruct((B, N * K, 1), jnp.float32),
            jax.ShapeDtypeStruct((B, N, D_OBJ), jnp.float32),
            jax.ShapeDtypeStruct((B, N, 1), jnp.float32),
        ],
    )(obj, g1, ub, maskf, tabg, dk, sp_w3, sp_b2, pr_w3, pr_b2, sc_w,
      vl_w, vl_b2, fu_w1a, langc, fu_b12, fu_w2, fu_b22, cl_w, cl_b2)


def kernel(points, object_mask, text_features, centers, sizes, pe_w1, pe_b1,
           pe_w2, pe_b2, le_w1, le_b1, le_w2, le_b2, sp_w, sp_b, pr_w, pr_b,
           sc_w, sc_b, vl_w, vl_b, fu_w1, fu_b1, fu_w2, fu_b2, cl_w, cl_b):
    maskf = object_mask.astype(jnp.float32)[:, :, None]
    maskf_row = object_mask.astype(jnp.float32)[:, None, :]
    centers_t = jnp.transpose(centers, (0, 2, 1))

    obj, g1, tab, ub, nbr, nbrg, dval, lang, langc = _encode(
        points, centers, sizes, maskf, centers_t, maskf_row,
        pe_w1, pe_b1[None, :], pe_w2, pe_b2[None, :],
        pr_w[0:D_OBJ], pr_w[D_OBJ:2 * D_OBJ], sp_w,
        text_features, le_w1, le_b1[None, :], le_w2, le_b2[None, :],
        fu_w1[D_OBJ:])

    tabg = _sc_gather(nbrg.reshape(-1), tab.reshape(B * N, D_RH))

    wgt2, enhanced, logits3 = _relation(
        obj, g1, ub, maskf, tabg.reshape(B, N * K, D_RH),
        dval.reshape(B, N * K, 1), sp_w[3:4], sp_b[None, :],
        pr_w[2 * D_OBJ:], pr_b[None, :], sc_w, vl_w, vl_b[None, :],
        fu_w1[0:D_OBJ], langc[:, None, :], fu_b1[None, :], fu_w2,
        fu_b2[None, :],
        cl_w, cl_b[None, :])

    wgt = wgt2.reshape(B, N, K)
    logits = logits3[:, :, 0]
    return (logits, enhanced, obj, lang, wgt, nbr)
